# Initial kernel scaffold; baseline (speedup 1.0000x reference)
import jax
import jax.numpy as jnp
from jax.experimental import pallas as pl


def kernel(env_sequence, edge_attr, initial_lice, params, edge_index, add_noise):
    raise NotImplementedError("write your pallas kernel here")


# speedup vs baseline: 2.3501x; 2.3501x over previous
import functools

import jax
import jax.numpy as jnp
from jax.experimental import pallas as pl

N = 50000
E = 800000
T = 8
HID = 64
ENV = 5
FEAT = 3

BN = 1000
BE = 8000


def _softplus(x):
    return jnp.maximum(x, 0.0) + jnp.log1p(jnp.exp(-jnp.abs(x)))


def _sigmoid(x):
    return 1.0 / (1.0 + jnp.exp(-x))


def _efproj_body(ea_ref, w1_ref, b1_ref, wc_ref, bc_ref, out_ref):
    ea = ea_ref[...]
    e1 = jnp.maximum(jnp.dot(ea, w1_ref[...].T,
                             preferred_element_type=jnp.float32) + b1_ref[...], 0.0)
    out_ref[...] = jnp.dot(e1, wc_ref[...].T,
                           preferred_element_type=jnp.float32) + bc_ref[...]


def _efproj(edge_attr, w1, b1, wc, bc):
    grid = E // BE
    return pl.pallas_call(
        _efproj_body,
        grid=(grid,),
        in_specs=[
            pl.BlockSpec((BE, 4), lambda i: (i, 0)),
            pl.BlockSpec((HID, 4), lambda i: (0, 0)),
            pl.BlockSpec((1, HID), lambda i: (0, 0)),
            pl.BlockSpec((HID, HID), lambda i: (0, 0)),
            pl.BlockSpec((1, HID), lambda i: (0, 0)),
        ],
        out_specs=pl.BlockSpec((BE, HID), lambda i: (i, 0)),
        out_shape=jax.ShapeDtypeStruct((E, HID), jnp.float32),
    )(edge_attr, w1, b1, wc, bc)


def _node_body(env_ref, h_ref, lice_ref, raw_ref, asum_ref,
               en_w1, en_b1, en_w2, en_b2, en_w3, en_b3,
               fec_w1, fec_b1, fec_w2, fec_b2, fec_w3, fec_b3,
               gru_wih, gru_bih, gru_whh, gru_bhh,
               de_w1, de_b1, de_w2, de_b2,
               at_w1b, sc_ref,
               h_out, lice_out, hw_out, g_out):
    env_t = env_ref[...]
    h = h_ref[...]
    lice = lice_ref[...]
    beta = sc_ref[0, 0]
    temp_sens = sc_ref[0, 1]

    tn = (env_t[:, 0:1] - 10.0) / 5.0
    f = jnp.maximum(tn * fec_w1[...].T + fec_b1[...], 0.0)
    f = jnp.maximum(jnp.dot(f, fec_w2[...].T,
                            preferred_element_type=jnp.float32) + fec_b2[...], 0.0)
    fec = _softplus(jnp.sum(f * fec_w3[...], axis=1, keepdims=True)
                    + fec_b3[...])

    pressure = (beta * raw_ref[...]) / (asum_ref[...] + 1e-8)

    x = jnp.maximum(jnp.dot(env_t, en_w1[...].T,
                            preferred_element_type=jnp.float32) + en_b1[...], 0.0)
    x = jnp.maximum(jnp.dot(x, en_w2[...].T,
                            preferred_element_type=jnp.float32) + en_b2[...], 0.0)
    env_enc = jnp.dot(x, en_w3[...].T,
                      preferred_element_type=jnp.float32) + en_b3[...]

    din = jnp.concatenate([env_enc + pressure, env_t, lice], axis=-1)
    gi = jnp.dot(din, gru_wih[...].T,
                 preferred_element_type=jnp.float32) + gru_bih[...]
    gh = jnp.dot(h, gru_whh[...].T,
                 preferred_element_type=jnp.float32) + gru_bhh[...]
    i_r, i_z, i_n = gi[:, :HID], gi[:, HID:2 * HID], gi[:, 2 * HID:]
    h_r, h_z, h_n = gh[:, :HID], gh[:, HID:2 * HID], gh[:, 2 * HID:]
    r = _sigmoid(i_r + h_r)
    z = _sigmoid(i_z + h_z)
    n = jnp.tanh(i_n + r * h_n)
    h_new = (1.0 - z) * n + z * h

    d = jnp.maximum(jnp.dot(h_new, de_w1[...].T,
                            preferred_element_type=jnp.float32) + de_b1[...], 0.0)
    lice_base = _softplus(jnp.dot(d, de_w2[...].T,
                                  preferred_element_type=jnp.float32) + de_b2[...])
    lice_new = lice_base * (1.0 + temp_sens * (fec - 1.0))

    h_out[...] = h_new
    lice_out[...] = lice_new
    hw_out[...] = jnp.dot(h_new, at_w1b[...].T,
                          preferred_element_type=jnp.float32)
    g_out[...] = lice_new[:, 0:1] * h_new


def _node_step(env_t, h, lice, raw, asum, p, at_w1b, scalars):
    grid = N // BN
    full = lambda r, c: pl.BlockSpec((r, c), lambda i: (0, 0))
    row = lambda c: pl.BlockSpec((BN, c), lambda i: (i, 0))
    return pl.pallas_call(
        _node_body,
        grid=(grid,),
        in_specs=[
            row(ENV), row(HID), row(FEAT), row(HID), row(1),
            full(HID, ENV), full(1, HID), full(HID, HID), full(1, HID),
            full(HID, HID), full(1, HID),
            full(32, 1), full(1, 32), full(32, 32), full(1, 32),
            full(1, 32), full(1, 1),
            full(3 * HID, HID + ENV + FEAT), full(1, 3 * HID),
            full(3 * HID, HID), full(1, 3 * HID),
            full(HID, HID), full(1, HID), full(FEAT, HID), full(1, FEAT),
            full(HID, HID), full(1, 2),
        ],
        out_specs=[row(HID), row(FEAT), row(HID), row(HID)],
        out_shape=[
            jax.ShapeDtypeStruct((N, HID), jnp.float32),
            jax.ShapeDtypeStruct((N, FEAT), jnp.float32),
            jax.ShapeDtypeStruct((N, HID), jnp.float32),
            jax.ShapeDtypeStruct((N, HID), jnp.float32),
        ],
    )(env_t, h, lice, raw, asum,
      p['en_w1'], p['en_b1'][None], p['en_w2'], p['en_b2'][None],
      p['en_w3'], p['en_b3'][None],
      p['fec_w1'], p['fec_b1'][None], p['fec_w2'], p['fec_b2'][None],
      p['fec_w3'], p['fec_b3'][None],
      p['gru_wih'], p['gru_bih'][None], p['gru_whh'], p['gru_bhh'][None],
      p['de_w1'], p['de_b1'][None], p['de_w2'], p['de_b2'][None],
      at_w1b, scalars)


def kernel(env_sequence, edge_attr, initial_lice, params, edge_index, add_noise):
    p = params
    src = edge_index[0]
    dst = edge_index[1]
    beta = jnp.exp(p['log_beta'])
    scalars = jnp.stack([beta, p['temp_sensitivity']])[None]

    at_w1a = p['at_w1'][:, :HID]
    at_w1b = p['at_w1'][:, HID:]
    wc = at_w1a @ p['ee_w2']
    bc = (p['ee_b2'] @ at_w1a.T + p['at_b1'])[None]
    ef_proj = _efproj(edge_attr, p['ee_w1'], p['ee_b1'][None], wc, bc)
    w2row = p['at_w2'][0]
    b2 = p['at_b2'][0]

    h = jnp.zeros((N, HID), dtype=jnp.float32)
    lice = initial_lice
    hw = jnp.zeros((N, HID), dtype=jnp.float32)
    g = jnp.zeros((N, HID), dtype=jnp.float32)

    traj = []
    for t in range(T):
        env_t = env_sequence[t]
        if t == 0:
            raw = jnp.zeros((N, HID), dtype=jnp.float32)
            asum = jnp.ones((N, 1), dtype=jnp.float32)
        else:
            logits = jnp.sum(jnp.maximum(ef_proj + hw[src], 0.0) * w2row,
                             axis=-1) + b2
            amax = jax.ops.segment_max(logits, dst, num_segments=N)
            aexp = jnp.exp(logits - amax[dst])
            asum = jax.ops.segment_sum(aexp, dst, num_segments=N)[:, None]
            raw = jax.ops.segment_sum(aexp[:, None] * g[src], dst,
                                      num_segments=N)
        h, lice, hw, g = _node_step(env_t, h, lice, raw, asum, p, at_w1b,
                                    scalars)
        traj.append(lice)
    return jnp.stack(traj)


# speedup vs baseline: 8.9394x; 3.8038x over previous
import functools

import jax
import jax.numpy as jnp
from jax import lax
from jax.experimental import pallas as pl
from jax.experimental.pallas import tpu as pltpu
from jax.experimental.pallas import tpu_sc as plsc

N = 50000
E = 800000
T = 8
HID = 64
ENV = 5
FEAT = 3

NC, NS, L = 2, 16, 16
NW = NC * NS
EP = 819200
EW1 = EP // NW
K1 = 256
C1 = EW1 // K1
G1 = K1 // L
EW2 = EP // NS
K2 = 256
C2 = EW2 // K2
N2 = 50048
NSL = N2 // NS
NEG = -3.0e38

BN = 1000
BE = 8192


def _softplus(x):
    return jnp.maximum(x, 0.0) + jnp.log1p(jnp.exp(-jnp.abs(x)))


def _sigmoid(x):
    return 1.0 / (1.0 + jnp.exp(-x))


def _take(x, i):
    dnums = lax.GatherDimensionNumbers(offset_dims=(),
                                       collapsed_slice_dims=(0,),
                                       start_index_map=(0,))
    return lax.gather(x, i[:, None], dnums, (1,),
                      mode=lax.GatherScatterMode.PROMISE_IN_BOUNDS)


def _seg_max_all(keys, vals):
    idx = jnp.arange(L, dtype=jnp.int32)
    v = vals
    for k in (1, 2, 4, 8):
        p = jnp.maximum(idx - k, 0)
        ok = (idx >= k) & (_take(keys, p) == keys)
        v = jnp.where(ok, jnp.maximum(v, _take(v, p)), v)
    for k in (1, 2, 4, 8):
        nx = jnp.minimum(idx + k, L - 1)
        ok = (idx <= L - 1 - k) & (_take(keys, nx) == keys)
        v = jnp.where(ok, jnp.maximum(v, _take(v, nx)), v)
    return v


def _seg_sum_all(keys, vals):
    idx = jnp.arange(L, dtype=jnp.int32)
    v = vals
    for k in (1, 2, 4, 8):
        p = jnp.maximum(idx - k, 0)
        ok = (idx >= k) & (_take(keys, p) == keys)
        v = v + jnp.where(ok, _take(v, p), 0.0)
    for k in (1, 2, 4, 8):
        nx = jnp.minimum(idx + k, L - 1)
        ok = (idx <= L - 1 - k) & (_take(keys, nx) == keys)
        v = jnp.where(ok, jnp.maximum(v, _take(v, nx)), v)
    return v


def _efproj_body(ea_ref, w1_ref, b1_ref, wc_ref, bc_ref, out_ref):
    ea = ea_ref[...]
    e1 = jnp.maximum(jnp.dot(ea, w1_ref[...].T,
                             preferred_element_type=jnp.float32) + b1_ref[...], 0.0)
    out_ref[...] = jnp.dot(e1, wc_ref[...].T,
                           preferred_element_type=jnp.float32) + bc_ref[...]


def _efproj(edge_attr_p, w1, b1, wc, bc):
    return pl.pallas_call(
        _efproj_body,
        grid=(EP // BE,),
        in_specs=[
            pl.BlockSpec((BE, 4), lambda i: (i, 0)),
            pl.BlockSpec((HID, 4), lambda i: (0, 0)),
            pl.BlockSpec((1, HID), lambda i: (0, 0)),
            pl.BlockSpec((HID, HID), lambda i: (0, 0)),
            pl.BlockSpec((1, HID), lambda i: (0, 0)),
        ],
        out_specs=pl.BlockSpec((BE, HID), lambda i: (i, 0)),
        out_shape=jax.ShapeDtypeStruct((EP, HID), jnp.float32),
    )(edge_attr_p, w1, b1, wc, bc)


def _sc_mesh():
    return plsc.VectorSubcoreMesh(core_axis_name="c", subcore_axis_name="s")


_SC_PARAMS = pltpu.CompilerParams(needs_layout_passes=False,
                                  use_tc_tiling_on_sc=False)


def _pass1(src2d, dst2d, ef, hw, aux):
    @functools.partial(
        pl.kernel,
        out_type=[jax.ShapeDtypeStruct((EP,), jnp.float32),
                  jax.ShapeDtypeStruct((NW, N2), jnp.float32)],
        mesh=_sc_mesh(),
        compiler_params=_SC_PARAMS,
        scratch_types=[
            pltpu.VMEM((2, 128), jnp.int32),
            pltpu.VMEM((2, 128), jnp.int32),
            pltpu.VMEM((K1, HID), jnp.float32),
            pltpu.VMEM((K1,), jnp.float32),
            pltpu.VMEM((N2,), jnp.float32),
            pltpu.VMEM((80,), jnp.float32),
            pltpu.SemaphoreType.DMA,
        ],
    )
    def k(src_hbm, dst_hbm, ef_hbm, hw_hbm, aux_hbm, logits_hbm, amax_hbm,
          srcb, dstb, efb, logb, amaxt, auxv, sem):
        c = lax.axis_index("c")
        s = lax.axis_index("s")
        wid = s * NC + c
        base = wid * EW1
        pltpu.sync_copy(aux_hbm, auxv)
        w2v = [auxv[pl.ds(q * L, L)] for q in range(HID // L)]
        b2s = auxv[pl.ds(HID, L)][0]

        def initb(i, carry):
            amaxt[pl.ds(i * L, L)] = jnp.full((L,), NEG, jnp.float32)
            return carry
        lax.fori_loop(0, N2 // L, initb, 0)

        def chunk(j, carry):
            off = base + j * K1
            r = wid * (EW1 // 128) + j * (K1 // 128)
            pltpu.sync_copy(src_hbm.at[pl.ds(r, 2), :], srcb)
            pltpu.sync_copy(dst_hbm.at[pl.ds(r, 2), :], dstb)
            pltpu.sync_copy(ef_hbm.at[pl.ds(off, K1), :], efb)
            for g in range(K1 // 128):
                pltpu.async_copy(hw_hbm.at[srcb.at[g]],
                                 efb.at[pl.ds(g * 128, 128), :],
                                 sem, add=True).wait()

            def grp(i, carry2):
                lanes = jnp.arange(L, dtype=jnp.int32)
                lv = jnp.zeros((L,), jnp.float32)
                for u in range(L):
                    e = i * L + u
                    t = jnp.maximum(efb[e, pl.ds(0, L)], 0.0) * w2v[0]
                    for q in range(1, HID // L):
                        t = t + jnp.maximum(efb[e, pl.ds(q * L, L)],
                                            0.0) * w2v[q]
                    lv = lv + jnp.where(lanes == u, jnp.sum(t), 0.0)
                logit = lv + b2s
                logb[pl.ds(i * L, L)] = logit
                row = i // 8
                col = (i % 8) * L
                dstv = dstb[row, pl.ds(col, L)]
                sd, sl = plsc.sort_key_val(dstv, logit)
                m = _seg_max_all(sd, sl)
                cur = plsc.load_gather(amaxt, [sd])
                plsc.store_scatter(amaxt, [sd], jnp.maximum(cur, m))
                return carry2
            lax.fori_loop(0, G1, grp, 0)
            pltpu.sync_copy(logb, logits_hbm.at[pl.ds(off, K1)])
            return carry
        lax.fori_loop(0, C1, chunk, 0)
        pltpu.sync_copy(amaxt, amax_hbm.at[wid])

    return k(src2d, dst2d, ef, hw, aux)


def _amax_combine_body(parts_ref, out_ref):
    out_ref[...] = jnp.max(parts_ref[...], axis=0, keepdims=True)


def _amax_combine(parts):
    return pl.pallas_call(
        _amax_combine_body,
        grid=(N2 // 128,),
        in_specs=[pl.BlockSpec((NW, 128), lambda i: (0, i))],
        out_specs=pl.BlockSpec((1, 128), lambda i: (0, i)),
        out_shape=jax.ShapeDtypeStruct((1, N2), jnp.float32),
    )(parts)


def _pass2(src2d, dst2d, logits, amax, g2, zeros32, zeros1):
    @functools.partial(
        pl.kernel,
        out_type=[jax.ShapeDtypeStruct((NC, N2, HID // 2), jnp.float32),
                  jax.ShapeDtypeStruct((NC, N2), jnp.float32)],
        mesh=_sc_mesh(),
        compiler_params=_SC_PARAMS,
        scratch_types=[
            pltpu.VMEM((2, 128), jnp.int32),
            pltpu.VMEM((2, 128), jnp.int32),
            pltpu.VMEM((K2,), jnp.float32),
            pltpu.VMEM((K2,), jnp.float32),
            pltpu.VMEM((K2,), jnp.float32),
            pltpu.VMEM((K2, HID // 2), jnp.float32),
            pltpu.VMEM_SHARED((N2, HID // 2), jnp.float32),
            pltpu.VMEM_SHARED((N2,), jnp.float32),
            pltpu.SemaphoreType.DMA,
        ],
    )
    def k(src_hbm, dst_hbm, logits_hbm, amax_hbm, g2_hbm, z32_hbm, z1_hbm,
          raw_hbm, asum_hbm,
          srcb, dstb, logb, amaxg, aexpb, growsb, accum, accum2, sem):
        c = lax.axis_index("c")
        s = lax.axis_index("s")
        pltpu.sync_copy(z32_hbm.at[pl.ds(s * NSL, NSL), :],
                        accum.at[pl.ds(s * NSL, NSL), :])
        pltpu.sync_copy(z1_hbm.at[pl.ds(s * NSL, NSL)],
                        accum2.at[pl.ds(s * NSL, NSL)])
        plsc.subcore_barrier()

        base = s * EW2

        def chunk(j, carry):
            off = base + j * K2
            r = s * (EW2 // 128) + j * (K2 // 128)
            pltpu.sync_copy(src_hbm.at[pl.ds(r, 2), :], srcb)
            pltpu.sync_copy(dst_hbm.at[pl.ds(r, 2), :], dstb)
            pltpu.sync_copy(logits_hbm.at[pl.ds(off, K2)], logb)

            def adj(i, carry2):
                row = i // 8
                col = (i % 8) * L
                srcb[row, pl.ds(col, L)] = srcb[row, pl.ds(col, L)] * 2 + c
                return carry2
            lax.fori_loop(0, 2 * (128 // L), adj, 0)
            for g in range(K2 // 128):
                pltpu.async_copy(g2_hbm.at[srcb.at[g]],
                                 growsb.at[pl.ds(g * 128, 128), :],
                                 sem).wait()
            for g in range(K2 // 128):
                pltpu.async_copy(amax_hbm.at[dstb.at[g]],
                                 amaxg.at[pl.ds(g * 128, 128)],
                                 sem).wait()

            def grp(i, carry2):
                logv = logb[pl.ds(i * L, L)]
                am = amaxg[pl.ds(i * L, L)]
                aexpb[pl.ds(i * L, L)] = jnp.exp(logv - am)
                return carry2
            lax.fori_loop(0, K2 // L, grp, 0)

            def rower(i, carry2):
                axv = aexpb[pl.ds(i * L, L)]
                for u in range(L):
                    e = i * L + u
                    sc = axv[u]
                    growsb[e, pl.ds(0, L)] = growsb[e, pl.ds(0, L)] * sc
                    growsb[e, pl.ds(L, L)] = growsb[e, pl.ds(L, L)] * sc
                return carry2
            lax.fori_loop(0, K2 // L, rower, 0)
            for g in range(K2 // 128):
                pltpu.sync_copy(growsb.at[pl.ds(g * 128, 128), :],
                                accum.at[dstb.at[g]], add=True)
            for g in range(K2 // 128):
                pltpu.sync_copy(aexpb.at[pl.ds(g * 128, 128)],
                                accum2.at[dstb.at[g]], add=True)
            return carry
        lax.fori_loop(0, C2, chunk, 0)
        plsc.subcore_barrier()

        pltpu.sync_copy(accum.at[pl.ds(s * NSL, NSL), :],
                        raw_hbm.at[c].at[pl.ds(s * NSL, NSL), :])
        pltpu.sync_copy(accum2.at[pl.ds(s * NSL, NSL)],
                        asum_hbm.at[c].at[pl.ds(s * NSL, NSL)])

    return k(src2d, dst2d, logits, amax, g2, zeros32, zeros1)


def _node_body(env_ref, h_ref, lice_ref, rawa_ref, rawb_ref, asum_ref,
               en_w1, en_b1, en_w2, en_b2, en_w3, en_b3,
               fec_w1, fec_b1, fec_w2, fec_b2, fec_w3, fec_b3,
               gru_wih, gru_bih, gru_whh, gru_bhh,
               de_w1, de_b1, de_w2, de_b2,
               at_w1b, sc_ref,
               h_out, lice_out, hw_out, g_out):
    env_t = env_ref[...]
    h = h_ref[...]
    lice = lice_ref[...]
    beta = sc_ref[0, 0]
    temp_sens = sc_ref[0, 1]

    tn = (env_t[:, 0:1] - 10.0) / 5.0
    f = jnp.maximum(tn * fec_w1[...].T + fec_b1[...], 0.0)
    f = jnp.maximum(jnp.dot(f, fec_w2[...].T,
                            preferred_element_type=jnp.float32) + fec_b2[...], 0.0)
    fec = _softplus(jnp.sum(f * fec_w3[...], axis=1, keepdims=True)
                    + fec_b3[...])

    asum = asum_ref[...]
    raw = jnp.concatenate([rawa_ref[...], rawb_ref[...]], axis=-1)
    pressure = (beta * raw) / (asum + 1e-8)

    x = jnp.maximum(jnp.dot(env_t, en_w1[...].T,
                            preferred_element_type=jnp.float32) + en_b1[...], 0.0)
    x = jnp.maximum(jnp.dot(x, en_w2[...].T,
                            preferred_element_type=jnp.float32) + en_b2[...], 0.0)
    env_enc = jnp.dot(x, en_w3[...].T,
                      preferred_element_type=jnp.float32) + en_b3[...]

    din = jnp.concatenate([env_enc + pressure, env_t, lice], axis=-1)
    gi = jnp.dot(din, gru_wih[...].T,
                 preferred_element_type=jnp.float32) + gru_bih[...]
    gh = jnp.dot(h, gru_whh[...].T,
                 preferred_element_type=jnp.float32) + gru_bhh[...]
    i_r, i_z, i_n = gi[:, :HID], gi[:, HID:2 * HID], gi[:, 2 * HID:]
    h_r, h_z, h_n = gh[:, :HID], gh[:, HID:2 * HID], gh[:, 2 * HID:]
    r = _sigmoid(i_r + h_r)
    z = _sigmoid(i_z + h_z)
    n = jnp.tanh(i_n + r * h_n)
    h_new = (1.0 - z) * n + z * h

    d = jnp.maximum(jnp.dot(h_new, de_w1[...].T,
                            preferred_element_type=jnp.float32) + de_b1[...], 0.0)
    lice_base = _softplus(jnp.dot(d, de_w2[...].T,
                                  preferred_element_type=jnp.float32) + de_b2[...])
    lice_new = lice_base * (1.0 + temp_sens * (fec - 1.0))

    h_out[...] = h_new
    lice_out[...] = lice_new
    hw_out[...] = jnp.dot(h_new, at_w1b[...].T,
                          preferred_element_type=jnp.float32)
    g_out[...] = lice_new[:, 0:1] * h_new


def _node_step(env_t, h, lice, rawa, rawb, asum, p, at_w1b, scalars):
    grid = N // BN
    full = lambda r, c: pl.BlockSpec((r, c), lambda i: (0, 0))
    row = lambda c: pl.BlockSpec((BN, c), lambda i: (i, 0))
    return pl.pallas_call(
        _node_body,
        grid=(grid,),
        in_specs=[
            row(ENV), row(HID), row(FEAT), row(HID // 2), row(HID // 2),
            row(1),
            full(HID, ENV), full(1, HID), full(HID, HID), full(1, HID),
            full(HID, HID), full(1, HID),
            full(32, 1), full(1, 32), full(32, 32), full(1, 32),
            full(1, 32), full(1, 1),
            full(3 * HID, HID + ENV + FEAT), full(1, 3 * HID),
            full(3 * HID, HID), full(1, 3 * HID),
            full(HID, HID), full(1, HID), full(FEAT, HID), full(1, FEAT),
            full(HID, HID), full(1, 2),
        ],
        out_specs=[row(HID), row(FEAT), row(HID), row(HID)],
        out_shape=[
            jax.ShapeDtypeStruct((N, HID), jnp.float32),
            jax.ShapeDtypeStruct((N, FEAT), jnp.float32),
            jax.ShapeDtypeStruct((N, HID), jnp.float32),
            jax.ShapeDtypeStruct((N, HID), jnp.float32),
        ],
    )(env_t, h, lice, rawa, rawb, asum,
      p['en_w1'], p['en_b1'][None], p['en_w2'], p['en_b2'][None],
      p['en_w3'], p['en_b3'][None],
      p['fec_w1'], p['fec_b1'][None], p['fec_w2'], p['fec_b2'][None],
      p['fec_w3'], p['fec_b3'][None],
      p['gru_wih'], p['gru_bih'][None], p['gru_whh'], p['gru_bhh'][None],
      p['de_w1'], p['de_b1'][None], p['de_w2'], p['de_b2'][None],
      at_w1b, scalars)


def kernel(env_sequence, edge_attr, initial_lice, params, edge_index, add_noise):
    p = params
    beta = jnp.exp(p['log_beta'])
    scalars = jnp.stack([beta, p['temp_sensitivity']])[None]

    at_w1a = p['at_w1'][:, :HID]
    at_w1b = p['at_w1'][:, HID:]
    wc = at_w1a @ p['ee_w2']
    bc = (p['ee_b2'] @ at_w1a.T + p['at_b1'])[None]
    aux = jnp.concatenate([p['at_w2'][0], p['at_b2'],
                           jnp.zeros((15,), jnp.float32)])

    src_p = jnp.pad(edge_index[0], (0, EP - E)).reshape(EP // 128, 128)
    dst_p = jnp.pad(edge_index[1], (0, EP - E),
                    constant_values=N).reshape(EP // 128, 128)
    ea_p = jnp.pad(edge_attr, ((0, EP - E), (0, 0)))
    ef_proj = _efproj(ea_p, p['ee_w1'], p['ee_b1'][None], wc, bc)

    h = jnp.zeros((N, HID), dtype=jnp.float32)
    lice = initial_lice
    hw = jnp.zeros((N, HID), dtype=jnp.float32)
    g = jnp.zeros((N, HID), dtype=jnp.float32)
    zeros32 = jnp.zeros((N2, HID // 2), dtype=jnp.float32)
    zeros1 = jnp.zeros((N2,), dtype=jnp.float32)

    traj = []
    for t in range(T):
        env_t = env_sequence[t]
        if t == 0:
            rawa = jnp.zeros((N2, HID // 2), dtype=jnp.float32)
            rawb = rawa
            asum = jnp.zeros((N2, 1), dtype=jnp.float32)
        else:
            logits, amax_parts = _pass1(src_p, dst_p, ef_proj, hw, aux)
            amax = _amax_combine(amax_parts).reshape(N2)
            g2 = g.reshape(2 * N, HID // 2)
            raw, asum_out = _pass2(src_p, dst_p, logits, amax, g2,
                                   zeros32, zeros1)
            asum = asum_out[0][:, None]
            rawa, rawb = raw[0], raw[1]
        h, lice, hw, g = _node_step(env_t, h, lice, rawa, rawb, asum, p,
                                    at_w1b, scalars)
        traj.append(lice)
    return jnp.stack(traj)


# speedup vs baseline: 12.5538x; 1.4043x over previous
import functools

import jax
import jax.numpy as jnp
from jax import lax
from jax.experimental import pallas as pl
from jax.experimental.pallas import tpu as pltpu
from jax.experimental.pallas import tpu_sc as plsc

N = 50000
E = 800000
T = 8
HID = 64
ENV = 5
FEAT = 3

NC, NS, L = 2, 16, 16
NW = NC * NS
EP = 819200
EW1 = EP // NW
K1 = 512
C1 = EW1 // K1
K1B = 1024
C1B = EW1 // K1B
EW2 = EP // NS
K2 = 256
C2 = EW2 // K2
N2 = 50048
NSL = N2 // NS
NEG = -3.0e38

BN = 1000
BE = 8192


def _softplus(x):
    return jnp.maximum(x, 0.0) + jnp.log1p(jnp.exp(-jnp.abs(x)))


def _sigmoid(x):
    return 1.0 / (1.0 + jnp.exp(-x))


def _take(x, i):
    dnums = lax.GatherDimensionNumbers(offset_dims=(),
                                       collapsed_slice_dims=(0,),
                                       start_index_map=(0,))
    return lax.gather(x, i[:, None], dnums, (1,),
                      mode=lax.GatherScatterMode.PROMISE_IN_BOUNDS)


def _seg_max_all(keys, vals):
    idx = jnp.arange(L, dtype=jnp.int32)
    v = vals
    for k in (1, 2, 4, 8):
        p = jnp.maximum(idx - k, 0)
        ok = (idx >= k) & (_take(keys, p) == keys)
        v = jnp.where(ok, jnp.maximum(v, _take(v, p)), v)
    for k in (1, 2, 4, 8):
        nx = jnp.minimum(idx + k, L - 1)
        ok = (idx <= L - 1 - k) & (_take(keys, nx) == keys)
        v = jnp.where(ok, jnp.maximum(v, _take(v, nx)), v)
    return v


def _seg_sum_all(keys, vals):
    idx = jnp.arange(L, dtype=jnp.int32)
    v = vals
    for k in (1, 2, 4, 8):
        p = jnp.maximum(idx - k, 0)
        ok = (idx >= k) & (_take(keys, p) == keys)
        v = v + jnp.where(ok, _take(v, p), 0.0)
    for k in (1, 2, 4, 8):
        nx = jnp.minimum(idx + k, L - 1)
        ok = (idx <= L - 1 - k) & (_take(keys, nx) == keys)
        v = jnp.where(ok, jnp.maximum(v, _take(v, nx)), v)
    return v


def _efproj_body(ea_ref, w1_ref, b1_ref, wc_ref, bc_ref, out_ref):
    ea = ea_ref[...]
    e1 = jnp.maximum(jnp.dot(ea, w1_ref[...].T,
                             preferred_element_type=jnp.float32) + b1_ref[...], 0.0)
    out_ref[...] = jnp.dot(e1, wc_ref[...].T,
                           preferred_element_type=jnp.float32) + bc_ref[...]


def _efproj(edge_attr_p, w1, b1, wc, bc):
    return pl.pallas_call(
        _efproj_body,
        grid=(EP // BE,),
        in_specs=[
            pl.BlockSpec((BE, 4), lambda i: (i, 0)),
            pl.BlockSpec((HID, 4), lambda i: (0, 0)),
            pl.BlockSpec((1, HID), lambda i: (0, 0)),
            pl.BlockSpec((HID, HID), lambda i: (0, 0)),
            pl.BlockSpec((1, HID), lambda i: (0, 0)),
        ],
        out_specs=pl.BlockSpec((BE, HID), lambda i: (i, 0)),
        out_shape=jax.ShapeDtypeStruct((EP, HID), jnp.float32),
    )(edge_attr_p, w1, b1, wc, bc)


def _sc_mesh():
    return plsc.VectorSubcoreMesh(core_axis_name="c", subcore_axis_name="s")


_SC_PARAMS = pltpu.CompilerParams(needs_layout_passes=False,
                                  use_tc_tiling_on_sc=False)


def _pass1a(src2d, ef, hw, aux):
    NB = 3

    @functools.partial(
        pl.kernel,
        out_type=[jax.ShapeDtypeStruct((EP,), jnp.float32)],
        mesh=_sc_mesh(),
        compiler_params=_SC_PARAMS,
        scratch_types=(
            [pltpu.VMEM((K1 // 128, 128), jnp.int32) for _ in range(NB)] +
            [pltpu.VMEM((K1, HID), jnp.float32) for _ in range(NB)] +
            [pltpu.VMEM((K1,), jnp.float32) for _ in range(NB)] +
            [pltpu.VMEM((80,), jnp.float32)] +
            [pltpu.SemaphoreType.DMA for _ in range(4 * NB)]
        ),
    )
    def k(src_hbm, ef_hbm, hw_hbm, aux_hbm, logits_hbm, *bufs):
        srcbs = bufs[0:NB]
        efbs = bufs[NB:2 * NB]
        logbs = bufs[2 * NB:3 * NB]
        auxv = bufs[3 * NB]
        isems = bufs[3 * NB + 1:3 * NB + 1 + NB]
        esems = bufs[3 * NB + 1 + NB:3 * NB + 1 + 2 * NB]
        gsems = bufs[3 * NB + 1 + 2 * NB:3 * NB + 1 + 3 * NB]
        osems = bufs[3 * NB + 1 + 3 * NB:3 * NB + 1 + 4 * NB]
        c = lax.axis_index("c")
        s = lax.axis_index("s")
        wid = s * NC + c
        base = wid * EW1
        pltpu.sync_copy(aux_hbm, auxv)
        w2v = [auxv[pl.ds(q * L, L)] for q in range(HID // L)]
        b2s = auxv[pl.ds(HID, L)][0]

        def idx_desc(j, b):
            r = wid * (EW1 // 128) + j * (K1 // 128)
            return pltpu.make_async_copy(
                src_hbm.at[pl.ds(r, K1 // 128), :], srcbs[b], isems[b])

        def ef_desc(j, b):
            off = base + j * K1
            return pltpu.make_async_copy(
                ef_hbm.at[pl.ds(off, K1), :], efbs[b], esems[b])

        def g_start(b, g):
            pltpu.async_copy(hw_hbm.at[srcbs[b].at[g]],
                             efbs[b].at[pl.ds(g * 128, 128), :],
                             gsems[b], add=True)

        def g_wait(b, g):
            pltpu.make_async_copy(hw_hbm.at[srcbs[b].at[g]],
                                  efbs[b].at[pl.ds(g * 128, 128), :],
                                  gsems[b]).wait()

        def o_desc(j, b):
            off = base + j * K1
            return pltpu.make_async_copy(
                logbs[b], logits_hbm.at[pl.ds(off, K1)], osems[b])

        def fire_pre(j, b):
            idx_desc(j, b).start()
            ef_desc(j, b).start()

        def fire_gather(j, b):
            idx_desc(j, b).wait()
            ef_desc(j, b).wait()
            for g in range(K1 // 128):
                g_start(b, g)

        def do_compute(j, b):
            for g in range(K1 // 128):
                g_wait(b, g)

            def grp(i, carry2):
                lanes = jnp.arange(L, dtype=jnp.int32)
                lv = jnp.zeros((L,), jnp.float32)
                for u in range(L):
                    e = i * L + u
                    t = jnp.maximum(efbs[b][e, pl.ds(0, L)], 0.0) * w2v[0]
                    for q in range(1, HID // L):
                        t = t + jnp.maximum(efbs[b][e, pl.ds(q * L, L)],
                                            0.0) * w2v[q]
                    lv = lv + jnp.where(lanes == u, jnp.sum(t), 0.0)
                logbs[b][pl.ds(i * L, L)] = lv + b2s
                return carry2
            lax.fori_loop(0, K1 // L, grp, 0)
            o_desc(j, b).start()

        def step3(jj, carry):
            for u in range(3):
                j = jj * 3 + u
                b = u

                @pl.when(j < C1)
                def _():
                    fire_pre(j, b)
                    fire_gather(j, b)
                    do_compute(j, b)
                    o_desc(j, b).wait()
            return carry
        lax.fori_loop(0, (C1 + 2) // 3, step3, 0)

    return k(src2d, ef, hw, aux)


def _pass1b(dst2d, logits):
    @functools.partial(
        pl.kernel,
        out_type=[jax.ShapeDtypeStruct((NW, N2), jnp.float32)],
        mesh=_sc_mesh(),
        compiler_params=_SC_PARAMS,
        scratch_types=(
            [pltpu.VMEM((K1B // 128, 128), jnp.int32) for _ in range(2)] +
            [pltpu.VMEM((K1B,), jnp.float32) for _ in range(2)] +
            [pltpu.VMEM((N2,), jnp.float32)] +
            [pltpu.SemaphoreType.DMA for _ in range(4)]
        ),
    )
    def k(dst_hbm, logits_hbm, amax_hbm, db0, db1, lb0, lb1, amaxt,
          is0, is1, ls0, ls1):
        dbs, lbs = (db0, db1), (lb0, lb1)
        isems, lsems = (is0, is1), (ls0, ls1)
        c = lax.axis_index("c")
        s = lax.axis_index("s")
        wid = s * NC + c
        base = wid * EW1

        def initb(i, carry):
            amaxt[pl.ds(i * L, L)] = jnp.full((L,), NEG, jnp.float32)
            return carry
        lax.fori_loop(0, N2 // L, initb, 0)

        def idx_desc(j, b):
            r = wid * (EW1 // 128) + j * (K1B // 128)
            return pltpu.make_async_copy(
                dst_hbm.at[pl.ds(r, K1B // 128), :], dbs[b], isems[b])

        def log_desc(j, b):
            off = base + j * K1B
            return pltpu.make_async_copy(
                logits_hbm.at[pl.ds(off, K1B)], lbs[b], lsems[b])

        idx_desc(0, 0).start()
        log_desc(0, 0).start()

        def do_chunk(j, b):
            @pl.when(j + 1 < C1B)
            def _():
                idx_desc(j + 1, 1 - b).start()
                log_desc(j + 1, 1 - b).start()
            idx_desc(j, b).wait()
            log_desc(j, b).wait()

            def grp(i, carry2):
                row = i // 8
                col = (i % 8) * L
                dstv = dbs[b][row, pl.ds(col, L)]
                logit = lbs[b][pl.ds(i * L, L)]
                sd, sl = plsc.sort_key_val(dstv, logit)
                m = _seg_max_all(sd, sl)
                cur = plsc.load_gather(amaxt, [sd])
                plsc.store_scatter(amaxt, [sd], jnp.maximum(cur, m))
                return carry2
            lax.fori_loop(0, K1B // L, grp, 0)

        def step2(jj, carry):
            for u in range(2):
                j = jj * 2 + u

                @pl.when(j < C1B)
                def _():
                    do_chunk(j, u)
            return carry
        lax.fori_loop(0, (C1B + 1) // 2, step2, 0)
        pltpu.sync_copy(amaxt, amax_hbm.at[wid])

    return k(dst2d, logits)


def _amax_combine_body(parts_ref, out_ref):
    out_ref[...] = jnp.max(parts_ref[...], axis=0, keepdims=True)


def _amax_combine(parts):
    return pl.pallas_call(
        _amax_combine_body,
        grid=(N2 // 128,),
        in_specs=[pl.BlockSpec((NW, 128), lambda i: (0, i))],
        out_specs=pl.BlockSpec((1, 128), lambda i: (0, i)),
        out_shape=jax.ShapeDtypeStruct((1, N2), jnp.float32),
    )(parts)


def _pass2(src2d, dst2d, logits, amax, g2, zeros32, zeros1):
    NB = 3
    GG = K2 // 128

    @functools.partial(
        pl.kernel,
        out_type=[jax.ShapeDtypeStruct((NC, N2, HID // 2), jnp.float32),
                  jax.ShapeDtypeStruct((NC, N2), jnp.float32)],
        mesh=_sc_mesh(),
        compiler_params=_SC_PARAMS,
        scratch_types=(
            [pltpu.VMEM((GG, 128), jnp.int32) for _ in range(2 * NB)] +
            [pltpu.VMEM((K2,), jnp.float32) for _ in range(2 * NB)] +
            [pltpu.VMEM((K2, HID // 2), jnp.float32) for _ in range(NB)] +
            [pltpu.VMEM_SHARED((N2, HID // 2), jnp.float32),
             pltpu.VMEM_SHARED((N2,), jnp.float32)] +
            [pltpu.SemaphoreType.DMA for _ in range(3 * NB)]
        ),
    )
    def k(src_hbm, dst_hbm, logits_hbm, amax_hbm, g2_hbm, z32_hbm, z1_hbm,
          raw_hbm, asum_hbm, *bufs):
        srcbs = bufs[0:NB]
        dstbs = bufs[NB:2 * NB]
        logbs = bufs[2 * NB:3 * NB]
        amaxgs = bufs[3 * NB:4 * NB]
        growbs = bufs[4 * NB:5 * NB]
        accum = bufs[5 * NB]
        accum2 = bufs[5 * NB + 1]
        psems = bufs[5 * NB + 2:5 * NB + 2 + NB]
        gsems = bufs[5 * NB + 2 + NB:5 * NB + 2 + 2 * NB]
        ssems = bufs[5 * NB + 2 + 2 * NB:5 * NB + 2 + 3 * NB]
        c = lax.axis_index("c")
        s = lax.axis_index("s")
        pltpu.sync_copy(z32_hbm.at[pl.ds(s * NSL, NSL), :],
                        accum.at[pl.ds(s * NSL, NSL), :])
        pltpu.sync_copy(z1_hbm.at[pl.ds(s * NSL, NSL)],
                        accum2.at[pl.ds(s * NSL, NSL)])
        plsc.subcore_barrier()

        base = s * EW2

        def src_desc(j, b):
            r = s * (EW2 // 128) + j * GG
            return pltpu.make_async_copy(
                src_hbm.at[pl.ds(r, GG), :], srcbs[b], psems[b])

        def dst_desc(j, b):
            r = s * (EW2 // 128) + j * GG
            return pltpu.make_async_copy(
                dst_hbm.at[pl.ds(r, GG), :], dstbs[b], psems[b])

        def log_desc(j, b):
            off = base + j * K2
            return pltpu.make_async_copy(
                logits_hbm.at[pl.ds(off, K2)], logbs[b], psems[b])

        def grow_desc(b, g):
            return pltpu.make_async_copy(
                g2_hbm.at[srcbs[b].at[g]],
                growbs[b].at[pl.ds(g * 128, 128), :], gsems[b])

        def amax_desc(b, g):
            return pltpu.make_async_copy(
                amax_hbm.at[dstbs[b].at[g]],
                amaxgs[b].at[pl.ds(g * 128, 128)], gsems[b])

        def scat_row_desc(b, g):
            return pltpu.make_async_copy(
                growbs[b].at[pl.ds(g * 128, 128), :],
                accum.at[dstbs[b].at[g]], ssems[b])

        def scat_sum_desc(b, g):
            return pltpu.make_async_copy(
                logbs[b].at[pl.ds(g * 128, 128)],
                accum2.at[dstbs[b].at[g]], ssems[b])

        def scat_wait(b):
            for g in range(GG):
                scat_row_desc(b, g).wait()
                scat_sum_desc(b, g).wait()

        def fire_pre(j, b, first=False):
            if not first:
                @pl.when(j >= 3)
                def _():
                    scat_wait(b)
            src_desc(j, b).start()
            dst_desc(j, b).start()
            log_desc(j, b).start()

        def prep_gather(j, b):
            src_desc(j, b).wait()
            dst_desc(j, b).wait()
            log_desc(j, b).wait()

            def adj(i, carry2):
                row = i // 8
                col = (i % 8) * L
                srcbs[b][row, pl.ds(col, L)] = (
                    srcbs[b][row, pl.ds(col, L)] * 2 + c)
                return carry2
            lax.fori_loop(0, GG * (128 // L), adj, 0)
            for g in range(GG):
                grow_desc(b, g).start()
                amax_desc(b, g).start()

        def compute(j, b):
            for g in range(GG):
                grow_desc(b, g).wait()
                amax_desc(b, g).wait()

            def grp(i, carry2):
                logv = logbs[b][pl.ds(i * L, L)]
                am = amaxgs[b][pl.ds(i * L, L)]
                logbs[b][pl.ds(i * L, L)] = jnp.exp(logv - am)
                return carry2
            lax.fori_loop(0, K2 // L, grp, 0)

            def rower(i, carry2):
                axv = logbs[b][pl.ds(i * L, L)]
                for u in range(L):
                    e = i * L + u
                    sc = axv[u]
                    growbs[b][e, pl.ds(0, L)] = (
                        growbs[b][e, pl.ds(0, L)] * sc)
                    growbs[b][e, pl.ds(L, L)] = (
                        growbs[b][e, pl.ds(L, L)] * sc)
                return carry2
            lax.fori_loop(0, K2 // L, rower, 0)
            for g in range(GG):
                pltpu.async_copy(growbs[b].at[pl.ds(g * 128, 128), :],
                                 accum.at[dstbs[b].at[g]], ssems[b],
                                 add=True)
                pltpu.async_copy(logbs[b].at[pl.ds(g * 128, 128)],
                                 accum2.at[dstbs[b].at[g]], ssems[b],
                                 add=True)

        def step3(jj, carry):
            for u in range(3):
                j = jj * 3 + u
                b = u

                @pl.when(j < C2)
                def _():
                    fire_pre(j, b, first=True)
                    prep_gather(j, b)
                    compute(j, b)
                    scat_wait(b)
            return carry
        lax.fori_loop(0, (C2 + 2) // 3, step3, 0)
        plsc.subcore_barrier()

        pltpu.sync_copy(accum.at[pl.ds(s * NSL, NSL), :],
                        raw_hbm.at[c].at[pl.ds(s * NSL, NSL), :])
        pltpu.sync_copy(accum2.at[pl.ds(s * NSL, NSL)],
                        asum_hbm.at[c].at[pl.ds(s * NSL, NSL)])

    return k(src2d, dst2d, logits, amax, g2, zeros32, zeros1)


def _node_body(env_ref, h_ref, lice_ref, rawa_ref, rawb_ref, asum_ref,
               en_w1, en_b1, en_w2, en_b2, en_w3, en_b3,
               fec_w1, fec_b1, fec_w2, fec_b2, fec_w3, fec_b3,
               gru_wih, gru_bih, gru_whh, gru_bhh,
               de_w1, de_b1, de_w2, de_b2,
               at_w1b, sc_ref,
               h_out, lice_out, hw_out, g_out):
    env_t = env_ref[...]
    h = h_ref[...]
    lice = lice_ref[...]
    beta = sc_ref[0, 0]
    temp_sens = sc_ref[0, 1]

    tn = (env_t[:, 0:1] - 10.0) / 5.0
    f = jnp.maximum(tn * fec_w1[...].T + fec_b1[...], 0.0)
    f = jnp.maximum(jnp.dot(f, fec_w2[...].T,
                            preferred_element_type=jnp.float32) + fec_b2[...], 0.0)
    fec = _softplus(jnp.sum(f * fec_w3[...], axis=1, keepdims=True)
                    + fec_b3[...])

    asum = asum_ref[...]
    raw = jnp.concatenate([rawa_ref[...], rawb_ref[...]], axis=-1)
    pressure = (beta * raw) / (asum + 1e-8)

    x = jnp.maximum(jnp.dot(env_t, en_w1[...].T,
                            preferred_element_type=jnp.float32) + en_b1[...], 0.0)
    x = jnp.maximum(jnp.dot(x, en_w2[...].T,
                            preferred_element_type=jnp.float32) + en_b2[...], 0.0)
    env_enc = jnp.dot(x, en_w3[...].T,
                      preferred_element_type=jnp.float32) + en_b3[...]

    din = jnp.concatenate([env_enc + pressure, env_t, lice], axis=-1)
    gi = jnp.dot(din, gru_wih[...].T,
                 preferred_element_type=jnp.float32) + gru_bih[...]
    gh = jnp.dot(h, gru_whh[...].T,
                 preferred_element_type=jnp.float32) + gru_bhh[...]
    i_r, i_z, i_n = gi[:, :HID], gi[:, HID:2 * HID], gi[:, 2 * HID:]
    h_r, h_z, h_n = gh[:, :HID], gh[:, HID:2 * HID], gh[:, 2 * HID:]
    r = _sigmoid(i_r + h_r)
    z = _sigmoid(i_z + h_z)
    n = jnp.tanh(i_n + r * h_n)
    h_new = (1.0 - z) * n + z * h

    d = jnp.maximum(jnp.dot(h_new, de_w1[...].T,
                            preferred_element_type=jnp.float32) + de_b1[...], 0.0)
    lice_base = _softplus(jnp.dot(d, de_w2[...].T,
                                  preferred_element_type=jnp.float32) + de_b2[...])
    lice_new = lice_base * (1.0 + temp_sens * (fec - 1.0))

    h_out[...] = h_new
    lice_out[...] = lice_new
    hw_out[...] = jnp.dot(h_new, at_w1b[...].T,
                          preferred_element_type=jnp.float32)
    g_out[...] = lice_new[:, 0:1] * h_new


def _node_step(env_t, h, lice, rawa, rawb, asum, p, at_w1b, scalars):
    grid = N // BN
    full = lambda r, c: pl.BlockSpec((r, c), lambda i: (0, 0))
    row = lambda c: pl.BlockSpec((BN, c), lambda i: (i, 0))
    return pl.pallas_call(
        _node_body,
        grid=(grid,),
        in_specs=[
            row(ENV), row(HID), row(FEAT), row(HID // 2), row(HID // 2),
            row(1),
            full(HID, ENV), full(1, HID), full(HID, HID), full(1, HID),
            full(HID, HID), full(1, HID),
            full(32, 1), full(1, 32), full(32, 32), full(1, 32),
            full(1, 32), full(1, 1),
            full(3 * HID, HID + ENV + FEAT), full(1, 3 * HID),
            full(3 * HID, HID), full(1, 3 * HID),
            full(HID, HID), full(1, HID), full(FEAT, HID), full(1, FEAT),
            full(HID, HID), full(1, 2),
        ],
        out_specs=[row(HID), row(FEAT), row(HID), row(HID)],
        out_shape=[
            jax.ShapeDtypeStruct((N, HID), jnp.float32),
            jax.ShapeDtypeStruct((N, FEAT), jnp.float32),
            jax.ShapeDtypeStruct((N, HID), jnp.float32),
            jax.ShapeDtypeStruct((N, HID), jnp.float32),
        ],
    )(env_t, h, lice, rawa, rawb, asum,
      p['en_w1'], p['en_b1'][None], p['en_w2'], p['en_b2'][None],
      p['en_w3'], p['en_b3'][None],
      p['fec_w1'], p['fec_b1'][None], p['fec_w2'], p['fec_b2'][None],
      p['fec_w3'], p['fec_b3'][None],
      p['gru_wih'], p['gru_bih'][None], p['gru_whh'], p['gru_bhh'][None],
      p['de_w1'], p['de_b1'][None], p['de_w2'], p['de_b2'][None],
      at_w1b, scalars)


def kernel(env_sequence, edge_attr, initial_lice, params, edge_index, add_noise):
    p = params
    beta = jnp.exp(p['log_beta'])
    scalars = jnp.stack([beta, p['temp_sensitivity']])[None]

    at_w1a = p['at_w1'][:, :HID]
    at_w1b = p['at_w1'][:, HID:]
    wc = at_w1a @ p['ee_w2']
    bc = (p['ee_b2'] @ at_w1a.T + p['at_b1'])[None]
    aux = jnp.concatenate([p['at_w2'][0], p['at_b2'],
                           jnp.zeros((15,), jnp.float32)])

    src_p = jnp.pad(edge_index[0], (0, EP - E)).reshape(EP // 128, 128)
    dst_p = jnp.pad(edge_index[1], (0, EP - E),
                    constant_values=N).reshape(EP // 128, 128)
    ea_p = jnp.pad(edge_attr, ((0, EP - E), (0, 0)))
    ef_proj = _efproj(ea_p, p['ee_w1'], p['ee_b1'][None], wc, bc)

    h = jnp.zeros((N, HID), dtype=jnp.float32)
    lice = initial_lice
    hw = jnp.zeros((N, HID), dtype=jnp.float32)
    g = jnp.zeros((N, HID), dtype=jnp.float32)
    zeros32 = jnp.zeros((N2, HID // 2), dtype=jnp.float32)
    zeros1 = jnp.zeros((N2,), dtype=jnp.float32)

    traj = []
    for t in range(T):
        env_t = env_sequence[t]
        if t == 0:
            rawa = jnp.zeros((N2, HID // 2), dtype=jnp.float32)
            rawb = rawa
            asum = jnp.zeros((N2, 1), dtype=jnp.float32)
        else:
            logits = _pass1a(src_p, ef_proj, hw, aux)[0]
            amax_parts = _pass1b(dst_p, logits)[0]
            amax = _amax_combine(amax_parts).reshape(N2)
            g2 = g.reshape(2 * N, HID // 2)
            raw, asum_out = _pass2(src_p, dst_p, logits, amax, g2,
                                   zeros32, zeros1)
            asum = asum_out[0][:, None]
            rawa, rawb = raw[0], raw[1]
        h, lice, hw, g = _node_step(env_t, h, lice, rawa, rawb, asum, p,
                                    at_w1b, scalars)
        traj.append(lice)
    return jnp.stack(traj)


# speedup vs baseline: 12.8279x; 1.0218x over previous
import functools

import jax
import jax.numpy as jnp
from jax import lax
from jax.experimental import pallas as pl
from jax.experimental.pallas import tpu as pltpu
from jax.experimental.pallas import tpu_sc as plsc

N = 50000
E = 800000
T = 8
HID = 64
ENV = 5
FEAT = 3

NC, NS, L = 2, 16, 16
NW = NC * NS
EP = 819200
EW1 = EP // NW
K1 = 512
C1 = EW1 // K1
K1B = 1024
C1B = EW1 // K1B
EW2 = EP // NS
K2 = 256
C2 = EW2 // K2
N2 = 50048
NSL = N2 // NS
NEG = -3.0e38

BN = 1000
BE = 8192


def _softplus(x):
    return jnp.maximum(x, 0.0) + jnp.log1p(jnp.exp(-jnp.abs(x)))


def _sigmoid(x):
    return 1.0 / (1.0 + jnp.exp(-x))


def _take(x, i):
    dnums = lax.GatherDimensionNumbers(offset_dims=(),
                                       collapsed_slice_dims=(0,),
                                       start_index_map=(0,))
    return lax.gather(x, i[:, None], dnums, (1,),
                      mode=lax.GatherScatterMode.PROMISE_IN_BOUNDS)


def _seg_max_all(keys, vals):
    idx = jnp.arange(L, dtype=jnp.int32)
    v = vals
    for k in (1, 2, 4, 8):
        p = jnp.maximum(idx - k, 0)
        ok = (idx >= k) & (_take(keys, p) == keys)
        v = jnp.where(ok, jnp.maximum(v, _take(v, p)), v)
    for k in (1, 2, 4, 8):
        nx = jnp.minimum(idx + k, L - 1)
        ok = (idx <= L - 1 - k) & (_take(keys, nx) == keys)
        v = jnp.where(ok, jnp.maximum(v, _take(v, nx)), v)
    return v


def _seg_sum_all(keys, vals):
    idx = jnp.arange(L, dtype=jnp.int32)
    v = vals
    for k in (1, 2, 4, 8):
        p = jnp.maximum(idx - k, 0)
        ok = (idx >= k) & (_take(keys, p) == keys)
        v = v + jnp.where(ok, _take(v, p), 0.0)
    for k in (1, 2, 4, 8):
        nx = jnp.minimum(idx + k, L - 1)
        ok = (idx <= L - 1 - k) & (_take(keys, nx) == keys)
        v = jnp.where(ok, jnp.maximum(v, _take(v, nx)), v)
    return v


def _efproj_body(ea_ref, w1_ref, b1_ref, wc_ref, bc_ref, out_ref):
    ea = ea_ref[...]
    e1 = jnp.maximum(jnp.dot(ea, w1_ref[...].T,
                             preferred_element_type=jnp.float32) + b1_ref[...], 0.0)
    out_ref[...] = jnp.dot(e1, wc_ref[...].T,
                           preferred_element_type=jnp.float32) + bc_ref[...]


def _efproj(edge_attr_p, w1, b1, wc, bc):
    return pl.pallas_call(
        _efproj_body,
        grid=(EP // BE,),
        in_specs=[
            pl.BlockSpec((BE, 4), lambda i: (i, 0)),
            pl.BlockSpec((HID, 4), lambda i: (0, 0)),
            pl.BlockSpec((1, HID), lambda i: (0, 0)),
            pl.BlockSpec((HID, HID), lambda i: (0, 0)),
            pl.BlockSpec((1, HID), lambda i: (0, 0)),
        ],
        out_specs=pl.BlockSpec((BE, HID), lambda i: (i, 0)),
        out_shape=jax.ShapeDtypeStruct((EP, HID), jnp.float32),
    )(edge_attr_p, w1, b1, wc, bc)


def _sc_mesh():
    return plsc.VectorSubcoreMesh(core_axis_name="c", subcore_axis_name="s")


_SC_PARAMS = pltpu.CompilerParams(needs_layout_passes=False,
                                  use_tc_tiling_on_sc=False)


def _pass1a(src2d, ef, hw, aux):
    NB = 3

    @functools.partial(
        pl.kernel,
        out_type=[jax.ShapeDtypeStruct((EP,), jnp.float32)],
        mesh=_sc_mesh(),
        compiler_params=_SC_PARAMS,
        scratch_types=(
            [pltpu.VMEM((K1 // 128, 128), jnp.int32) for _ in range(NB)] +
            [pltpu.VMEM((K1, HID), jnp.float32) for _ in range(NB)] +
            [pltpu.VMEM((K1,), jnp.float32) for _ in range(NB)] +
            [pltpu.VMEM((80,), jnp.float32)] +
            [pltpu.SemaphoreType.DMA for _ in range(4 * NB)]
        ),
    )
    def k(src_hbm, ef_hbm, hw_hbm, aux_hbm, logits_hbm, *bufs):
        srcbs = bufs[0:NB]
        efbs = bufs[NB:2 * NB]
        logbs = bufs[2 * NB:3 * NB]
        auxv = bufs[3 * NB]
        isems = bufs[3 * NB + 1:3 * NB + 1 + NB]
        esems = bufs[3 * NB + 1 + NB:3 * NB + 1 + 2 * NB]
        gsems = bufs[3 * NB + 1 + 2 * NB:3 * NB + 1 + 3 * NB]
        osems = bufs[3 * NB + 1 + 3 * NB:3 * NB + 1 + 4 * NB]
        c = lax.axis_index("c")
        s = lax.axis_index("s")
        wid = s * NC + c
        base = wid * EW1
        pltpu.sync_copy(aux_hbm, auxv)
        w2v = [auxv[pl.ds(q * L, L)] for q in range(HID // L)]
        b2s = auxv[pl.ds(HID, L)][0]

        def idx_desc(j, b):
            r = wid * (EW1 // 128) + j * (K1 // 128)
            return pltpu.make_async_copy(
                src_hbm.at[pl.ds(r, K1 // 128), :], srcbs[b], isems[b])

        def ef_desc(j, b):
            off = base + j * K1
            return pltpu.make_async_copy(
                ef_hbm.at[pl.ds(off, K1), :], efbs[b], esems[b])

        def g_start(b, g):
            pltpu.async_copy(hw_hbm.at[srcbs[b].at[g]],
                             efbs[b].at[pl.ds(g * 128, 128), :],
                             gsems[b], add=True)

        def g_wait(b, g):
            pltpu.make_async_copy(hw_hbm.at[srcbs[b].at[g]],
                                  efbs[b].at[pl.ds(g * 128, 128), :],
                                  gsems[b]).wait()

        def o_desc(j, b):
            off = base + j * K1
            return pltpu.make_async_copy(
                logbs[b], logits_hbm.at[pl.ds(off, K1)], osems[b])

        def fire_pre(j, b):
            idx_desc(j, b).start()
            ef_desc(j, b).start()

        def fire_gather(j, b):
            idx_desc(j, b).wait()
            ef_desc(j, b).wait()
            for g in range(K1 // 128):
                g_start(b, g)

        def do_compute(j, b):
            for g in range(K1 // 128):
                g_wait(b, g)

            def grp(i, carry2):
                lanes = jnp.arange(L, dtype=jnp.int32)
                lv = jnp.zeros((L,), jnp.float32)
                for u in range(L):
                    e = i * L + u
                    t = jnp.maximum(efbs[b][e, pl.ds(0, L)], 0.0) * w2v[0]
                    for q in range(1, HID // L):
                        t = t + jnp.maximum(efbs[b][e, pl.ds(q * L, L)],
                                            0.0) * w2v[q]
                    lv = lv + jnp.where(lanes == u, jnp.sum(t), 0.0)
                logbs[b][pl.ds(i * L, L)] = lv + b2s
                return carry2
            lax.fori_loop(0, K1 // L, grp, 0)
            o_desc(j, b).start()

        fire_pre(0, 0)

        def step3(jj, carry):
            for u in range(3):
                j = jj * 3 + u
                b = u

                @pl.when(j + 1 < C1)
                def _():
                    fire_pre(j + 1, (b + 1) % 3)

                @pl.when(j >= 3)
                def _():
                    o_desc(j - 3, b).wait()

                @pl.when(j < C1)
                def _():
                    fire_gather(j, b)
                    do_compute(j, b)
            return carry
        lax.fori_loop(0, (C1 + 2) // 3, step3, 0)
        for j in range(((C1 + 2) // 3) * 3 - 3, C1):
            o_desc(j, j % 3).wait()

    return k(src2d, ef, hw, aux)


def _pass1b(dst2d, logits):
    @functools.partial(
        pl.kernel,
        out_type=[jax.ShapeDtypeStruct((NW, N2), jnp.float32)],
        mesh=_sc_mesh(),
        compiler_params=_SC_PARAMS,
        scratch_types=(
            [pltpu.VMEM((K1B // 128, 128), jnp.int32) for _ in range(2)] +
            [pltpu.VMEM((K1B,), jnp.float32) for _ in range(2)] +
            [pltpu.VMEM((N2,), jnp.float32)] +
            [pltpu.SemaphoreType.DMA for _ in range(4)]
        ),
    )
    def k(dst_hbm, logits_hbm, amax_hbm, db0, db1, lb0, lb1, amaxt,
          is0, is1, ls0, ls1):
        dbs, lbs = (db0, db1), (lb0, lb1)
        isems, lsems = (is0, is1), (ls0, ls1)
        c = lax.axis_index("c")
        s = lax.axis_index("s")
        wid = s * NC + c
        base = wid * EW1

        def initb(i, carry):
            amaxt[pl.ds(i * L, L)] = jnp.full((L,), NEG, jnp.float32)
            return carry
        lax.fori_loop(0, N2 // L, initb, 0)

        def idx_desc(j, b):
            r = wid * (EW1 // 128) + j * (K1B // 128)
            return pltpu.make_async_copy(
                dst_hbm.at[pl.ds(r, K1B // 128), :], dbs[b], isems[b])

        def log_desc(j, b):
            off = base + j * K1B
            return pltpu.make_async_copy(
                logits_hbm.at[pl.ds(off, K1B)], lbs[b], lsems[b])

        idx_desc(0, 0).start()
        log_desc(0, 0).start()

        def do_chunk(j, b):
            @pl.when(j + 1 < C1B)
            def _():
                idx_desc(j + 1, 1 - b).start()
                log_desc(j + 1, 1 - b).start()
            idx_desc(j, b).wait()
            log_desc(j, b).wait()

            def grp(i, carry2):
                row = i // 8
                col = (i % 8) * L
                dstv = dbs[b][row, pl.ds(col, L)]
                logit = lbs[b][pl.ds(i * L, L)]
                sd, sl = plsc.sort_key_val(dstv, logit)
                m = _seg_max_all(sd, sl)
                cur = plsc.load_gather(amaxt, [sd])
                plsc.store_scatter(amaxt, [sd], jnp.maximum(cur, m))
                return carry2
            lax.fori_loop(0, K1B // L, grp, 0)

        def step2(jj, carry):
            for u in range(2):
                j = jj * 2 + u

                @pl.when(j < C1B)
                def _():
                    do_chunk(j, u)
            return carry
        lax.fori_loop(0, (C1B + 1) // 2, step2, 0)
        pltpu.sync_copy(amaxt, amax_hbm.at[wid])

    return k(dst2d, logits)


def _amax_combine_body(parts_ref, out_ref):
    out_ref[...] = jnp.max(parts_ref[...], axis=0, keepdims=True)


def _amax_combine(parts):
    return pl.pallas_call(
        _amax_combine_body,
        grid=(N2 // 128,),
        in_specs=[pl.BlockSpec((NW, 128), lambda i: (0, i))],
        out_specs=pl.BlockSpec((1, 128), lambda i: (0, i)),
        out_shape=jax.ShapeDtypeStruct((1, N2), jnp.float32),
    )(parts)


def _pass2(src2d, dst2d, logits, amax, g2, zeros32, zeros1):
    NB = 3
    GG = K2 // 128

    @functools.partial(
        pl.kernel,
        out_type=[jax.ShapeDtypeStruct((NC, N2, HID // 2), jnp.float32),
                  jax.ShapeDtypeStruct((NC, N2), jnp.float32)],
        mesh=_sc_mesh(),
        compiler_params=_SC_PARAMS,
        scratch_types=(
            [pltpu.VMEM((GG, 128), jnp.int32) for _ in range(2 * NB)] +
            [pltpu.VMEM((K2,), jnp.float32) for _ in range(2 * NB)] +
            [pltpu.VMEM((K2, HID // 2), jnp.float32) for _ in range(NB)] +
            [pltpu.VMEM_SHARED((N2, HID // 2), jnp.float32),
             pltpu.VMEM_SHARED((N2,), jnp.float32)] +
            [pltpu.SemaphoreType.DMA for _ in range(3 * NB)]
        ),
    )
    def k(src_hbm, dst_hbm, logits_hbm, amax_hbm, g2_hbm, z32_hbm, z1_hbm,
          raw_hbm, asum_hbm, *bufs):
        srcbs = bufs[0:NB]
        dstbs = bufs[NB:2 * NB]
        logbs = bufs[2 * NB:3 * NB]
        amaxgs = bufs[3 * NB:4 * NB]
        growbs = bufs[4 * NB:5 * NB]
        accum = bufs[5 * NB]
        accum2 = bufs[5 * NB + 1]
        psems = bufs[5 * NB + 2:5 * NB + 2 + NB]
        gsems = bufs[5 * NB + 2 + NB:5 * NB + 2 + 2 * NB]
        ssems = bufs[5 * NB + 2 + 2 * NB:5 * NB + 2 + 3 * NB]
        c = lax.axis_index("c")
        s = lax.axis_index("s")
        pltpu.sync_copy(z32_hbm.at[pl.ds(s * NSL, NSL), :],
                        accum.at[pl.ds(s * NSL, NSL), :])
        pltpu.sync_copy(z1_hbm.at[pl.ds(s * NSL, NSL)],
                        accum2.at[pl.ds(s * NSL, NSL)])
        plsc.subcore_barrier()

        base = s * EW2

        def src_desc(j, b):
            r = s * (EW2 // 128) + j * GG
            return pltpu.make_async_copy(
                src_hbm.at[pl.ds(r, GG), :], srcbs[b], psems[b])

        def dst_desc(j, b):
            r = s * (EW2 // 128) + j * GG
            return pltpu.make_async_copy(
                dst_hbm.at[pl.ds(r, GG), :], dstbs[b], psems[b])

        def log_desc(j, b):
            off = base + j * K2
            return pltpu.make_async_copy(
                logits_hbm.at[pl.ds(off, K2)], logbs[b], psems[b])

        def grow_desc(b, g):
            return pltpu.make_async_copy(
                g2_hbm.at[srcbs[b].at[g]],
                growbs[b].at[pl.ds(g * 128, 128), :], gsems[b])

        def amax_desc(b, g):
            return pltpu.make_async_copy(
                amax_hbm.at[dstbs[b].at[g]],
                amaxgs[b].at[pl.ds(g * 128, 128)], gsems[b])

        def scat_row_desc(b, g):
            return pltpu.make_async_copy(
                growbs[b].at[pl.ds(g * 128, 128), :],
                accum.at[dstbs[b].at[g]], ssems[b])

        def scat_sum_desc(b, g):
            return pltpu.make_async_copy(
                logbs[b].at[pl.ds(g * 128, 128)],
                accum2.at[dstbs[b].at[g]], ssems[b])

        def scat_wait(b):
            for g in range(GG):
                scat_row_desc(b, g).wait()
                scat_sum_desc(b, g).wait()

        def fire_pre(j, b, first=False):
            if not first:
                @pl.when(j >= 3)
                def _():
                    scat_wait(b)
            src_desc(j, b).start()
            dst_desc(j, b).start()
            log_desc(j, b).start()

        def prep_gather(j, b):
            src_desc(j, b).wait()
            dst_desc(j, b).wait()
            log_desc(j, b).wait()

            def adj(i, carry2):
                row = i // 8
                col = (i % 8) * L
                srcbs[b][row, pl.ds(col, L)] = (
                    srcbs[b][row, pl.ds(col, L)] * 2 + c)
                return carry2
            lax.fori_loop(0, GG * (128 // L), adj, 0)
            for g in range(GG):
                grow_desc(b, g).start()
                amax_desc(b, g).start()

        def compute(j, b):
            for g in range(GG):
                grow_desc(b, g).wait()
                amax_desc(b, g).wait()

            def grp(i, carry2):
                logv = logbs[b][pl.ds(i * L, L)]
                am = amaxgs[b][pl.ds(i * L, L)]
                logbs[b][pl.ds(i * L, L)] = jnp.exp(logv - am)
                return carry2
            lax.fori_loop(0, K2 // L, grp, 0)

            def rower(i, carry2):
                axv = logbs[b][pl.ds(i * L, L)]
                for u in range(L):
                    e = i * L + u
                    sc = axv[u]
                    growbs[b][e, pl.ds(0, L)] = (
                        growbs[b][e, pl.ds(0, L)] * sc)
                    growbs[b][e, pl.ds(L, L)] = (
                        growbs[b][e, pl.ds(L, L)] * sc)
                return carry2
            lax.fori_loop(0, K2 // L, rower, 0)
            for g in range(GG):
                pltpu.async_copy(growbs[b].at[pl.ds(g * 128, 128), :],
                                 accum.at[dstbs[b].at[g]], ssems[b],
                                 add=True)
                pltpu.async_copy(logbs[b].at[pl.ds(g * 128, 128)],
                                 accum2.at[dstbs[b].at[g]], ssems[b],
                                 add=True)

        def step3(jj, carry):
            for u in range(3):
                j = jj * 3 + u
                b = u

                @pl.when(j < C2)
                def _():
                    fire_pre(j, b, first=True)
                    prep_gather(j, b)
                    compute(j, b)
                    scat_wait(b)
            return carry
        lax.fori_loop(0, (C2 + 2) // 3, step3, 0)
        plsc.subcore_barrier()

        pltpu.sync_copy(accum.at[pl.ds(s * NSL, NSL), :],
                        raw_hbm.at[c].at[pl.ds(s * NSL, NSL), :])
        pltpu.sync_copy(accum2.at[pl.ds(s * NSL, NSL)],
                        asum_hbm.at[c].at[pl.ds(s * NSL, NSL)])

    return k(src2d, dst2d, logits, amax, g2, zeros32, zeros1)


def _node_body(env_ref, h_ref, lice_ref, rawa_ref, rawb_ref, asum_ref,
               en_w1, en_b1, en_w2, en_b2, en_w3, en_b3,
               fec_w1, fec_b1, fec_w2, fec_b2, fec_w3, fec_b3,
               gru_wih, gru_bih, gru_whh, gru_bhh,
               de_w1, de_b1, de_w2, de_b2,
               at_w1b, sc_ref,
               h_out, lice_out, hw_out, g_out):
    env_t = env_ref[...]
    h = h_ref[...]
    lice = lice_ref[...]
    beta = sc_ref[0, 0]
    temp_sens = sc_ref[0, 1]

    tn = (env_t[:, 0:1] - 10.0) / 5.0
    f = jnp.maximum(tn * fec_w1[...].T + fec_b1[...], 0.0)
    f = jnp.maximum(jnp.dot(f, fec_w2[...].T,
                            preferred_element_type=jnp.float32) + fec_b2[...], 0.0)
    fec = _softplus(jnp.sum(f * fec_w3[...], axis=1, keepdims=True)
                    + fec_b3[...])

    asum = asum_ref[...]
    raw = jnp.concatenate([rawa_ref[...], rawb_ref[...]], axis=-1)
    pressure = (beta * raw) / (asum + 1e-8)

    x = jnp.maximum(jnp.dot(env_t, en_w1[...].T,
                            preferred_element_type=jnp.float32) + en_b1[...], 0.0)
    x = jnp.maximum(jnp.dot(x, en_w2[...].T,
                            preferred_element_type=jnp.float32) + en_b2[...], 0.0)
    env_enc = jnp.dot(x, en_w3[...].T,
                      preferred_element_type=jnp.float32) + en_b3[...]

    din = jnp.concatenate([env_enc + pressure, env_t, lice], axis=-1)
    gi = jnp.dot(din, gru_wih[...].T,
                 preferred_element_type=jnp.float32) + gru_bih[...]
    gh = jnp.dot(h, gru_whh[...].T,
                 preferred_element_type=jnp.float32) + gru_bhh[...]
    i_r, i_z, i_n = gi[:, :HID], gi[:, HID:2 * HID], gi[:, 2 * HID:]
    h_r, h_z, h_n = gh[:, :HID], gh[:, HID:2 * HID], gh[:, 2 * HID:]
    r = _sigmoid(i_r + h_r)
    z = _sigmoid(i_z + h_z)
    n = jnp.tanh(i_n + r * h_n)
    h_new = (1.0 - z) * n + z * h

    d = jnp.maximum(jnp.dot(h_new, de_w1[...].T,
                            preferred_element_type=jnp.float32) + de_b1[...], 0.0)
    lice_base = _softplus(jnp.dot(d, de_w2[...].T,
                                  preferred_element_type=jnp.float32) + de_b2[...])
    lice_new = lice_base * (1.0 + temp_sens * (fec - 1.0))

    h_out[...] = h_new
    lice_out[...] = lice_new
    hw_out[...] = jnp.dot(h_new, at_w1b[...].T,
                          preferred_element_type=jnp.float32)
    g_out[...] = lice_new[:, 0:1] * h_new


def _node_step(env_t, h, lice, rawa, rawb, asum, p, at_w1b, scalars):
    grid = N // BN
    full = lambda r, c: pl.BlockSpec((r, c), lambda i: (0, 0))
    row = lambda c: pl.BlockSpec((BN, c), lambda i: (i, 0))
    return pl.pallas_call(
        _node_body,
        grid=(grid,),
        in_specs=[
            row(ENV), row(HID), row(FEAT), row(HID // 2), row(HID // 2),
            row(1),
            full(HID, ENV), full(1, HID), full(HID, HID), full(1, HID),
            full(HID, HID), full(1, HID),
            full(32, 1), full(1, 32), full(32, 32), full(1, 32),
            full(1, 32), full(1, 1),
            full(3 * HID, HID + ENV + FEAT), full(1, 3 * HID),
            full(3 * HID, HID), full(1, 3 * HID),
            full(HID, HID), full(1, HID), full(FEAT, HID), full(1, FEAT),
            full(HID, HID), full(1, 2),
        ],
        out_specs=[row(HID), row(FEAT), row(HID), row(HID)],
        out_shape=[
            jax.ShapeDtypeStruct((N, HID), jnp.float32),
            jax.ShapeDtypeStruct((N, FEAT), jnp.float32),
            jax.ShapeDtypeStruct((N, HID), jnp.float32),
            jax.ShapeDtypeStruct((N, HID), jnp.float32),
        ],
    )(env_t, h, lice, rawa, rawb, asum,
      p['en_w1'], p['en_b1'][None], p['en_w2'], p['en_b2'][None],
      p['en_w3'], p['en_b3'][None],
      p['fec_w1'], p['fec_b1'][None], p['fec_w2'], p['fec_b2'][None],
      p['fec_w3'], p['fec_b3'][None],
      p['gru_wih'], p['gru_bih'][None], p['gru_whh'], p['gru_bhh'][None],
      p['de_w1'], p['de_b1'][None], p['de_w2'], p['de_b2'][None],
      at_w1b, scalars)


def kernel(env_sequence, edge_attr, initial_lice, params, edge_index, add_noise):
    p = params
    beta = jnp.exp(p['log_beta'])
    scalars = jnp.stack([beta, p['temp_sensitivity']])[None]

    at_w1a = p['at_w1'][:, :HID]
    at_w1b = p['at_w1'][:, HID:]
    wc = at_w1a @ p['ee_w2']
    bc = (p['ee_b2'] @ at_w1a.T + p['at_b1'])[None]
    aux = jnp.concatenate([p['at_w2'][0], p['at_b2'],
                           jnp.zeros((15,), jnp.float32)])

    src_p = jnp.pad(edge_index[0], (0, EP - E)).reshape(EP // 128, 128)
    dst_p = jnp.pad(edge_index[1], (0, EP - E),
                    constant_values=N).reshape(EP // 128, 128)
    ea_p = jnp.pad(edge_attr, ((0, EP - E), (0, 0)))
    ef_proj = _efproj(ea_p, p['ee_w1'], p['ee_b1'][None], wc, bc)

    h = jnp.zeros((N, HID), dtype=jnp.float32)
    lice = initial_lice
    hw = jnp.zeros((N, HID), dtype=jnp.float32)
    g = jnp.zeros((N, HID), dtype=jnp.float32)
    zeros32 = jnp.zeros((N2, HID // 2), dtype=jnp.float32)
    zeros1 = jnp.zeros((N2,), dtype=jnp.float32)

    traj = []
    for t in range(T):
        env_t = env_sequence[t]
        if t == 0:
            rawa = jnp.zeros((N2, HID // 2), dtype=jnp.float32)
            rawb = rawa
            asum = jnp.zeros((N2, 1), dtype=jnp.float32)
        else:
            logits = _pass1a(src_p, ef_proj, hw, aux)[0]
            amax_parts = _pass1b(dst_p, logits)[0]
            amax = _amax_combine(amax_parts).reshape(N2)
            g2 = g.reshape(2 * N, HID // 2)
            raw, asum_out = _pass2(src_p, dst_p, logits, amax, g2,
                                   zeros32, zeros1)
            asum = asum_out[0][:, None]
            rawa, rawb = raw[0], raw[1]
        h, lice, hw, g = _node_step(env_t, h, lice, rawa, rawb, asum, p,
                                    at_w1b, scalars)
        traj.append(lice)
    return jnp.stack(traj)


# speedup vs baseline: 13.4745x; 1.0504x over previous
import functools

import jax
import jax.numpy as jnp
from jax import lax
from jax.experimental import pallas as pl
from jax.experimental.pallas import tpu as pltpu
from jax.experimental.pallas import tpu_sc as plsc

N = 50000
E = 800000
T = 8
HID = 64
ENV = 5
FEAT = 3

NC, NS, L = 2, 16, 16
NW = NC * NS
EP = 819200
EW1 = EP // NW
K1 = 512
C1 = EW1 // K1
K1B = 1024
C1B = EW1 // K1B
EW2 = EP // NS
K2 = 256
C2 = EW2 // K2
N2 = 50048
NSL = N2 // NS
NEG = -3.0e38

BN = 1000
BE = 8192


def _softplus(x):
    return jnp.maximum(x, 0.0) + jnp.log1p(jnp.exp(-jnp.abs(x)))


def _sigmoid(x):
    return 1.0 / (1.0 + jnp.exp(-x))


def _take(x, i):
    dnums = lax.GatherDimensionNumbers(offset_dims=(),
                                       collapsed_slice_dims=(0,),
                                       start_index_map=(0,))
    return lax.gather(x, i[:, None], dnums, (1,),
                      mode=lax.GatherScatterMode.PROMISE_IN_BOUNDS)


def _seg_max_all(keys, vals):
    idx = jnp.arange(L, dtype=jnp.int32)
    v = vals
    for k in (1, 2, 4, 8):
        p = jnp.maximum(idx - k, 0)
        ok = (idx >= k) & (_take(keys, p) == keys)
        v = jnp.where(ok, jnp.maximum(v, _take(v, p)), v)
    for k in (1, 2, 4, 8):
        nx = jnp.minimum(idx + k, L - 1)
        ok = (idx <= L - 1 - k) & (_take(keys, nx) == keys)
        v = jnp.where(ok, jnp.maximum(v, _take(v, nx)), v)
    return v


def _seg_sum_all(keys, vals):
    idx = jnp.arange(L, dtype=jnp.int32)
    v = vals
    for k in (1, 2, 4, 8):
        p = jnp.maximum(idx - k, 0)
        ok = (idx >= k) & (_take(keys, p) == keys)
        v = v + jnp.where(ok, _take(v, p), 0.0)
    for k in (1, 2, 4, 8):
        nx = jnp.minimum(idx + k, L - 1)
        ok = (idx <= L - 1 - k) & (_take(keys, nx) == keys)
        v = jnp.where(ok, jnp.maximum(v, _take(v, nx)), v)
    return v


def _efproj_body(ea_ref, w1_ref, b1_ref, wc_ref, bc_ref, out_ref):
    ea = ea_ref[...]
    e1 = jnp.maximum(jnp.dot(ea, w1_ref[...].T,
                             preferred_element_type=jnp.float32) + b1_ref[...], 0.0)
    out_ref[...] = jnp.dot(e1, wc_ref[...].T,
                           preferred_element_type=jnp.float32) + bc_ref[...]


def _efproj(edge_attr_p, w1, b1, wc, bc):
    return pl.pallas_call(
        _efproj_body,
        grid=(EP // BE,),
        in_specs=[
            pl.BlockSpec((BE, 4), lambda i: (i, 0)),
            pl.BlockSpec((HID, 4), lambda i: (0, 0)),
            pl.BlockSpec((1, HID), lambda i: (0, 0)),
            pl.BlockSpec((HID, HID), lambda i: (0, 0)),
            pl.BlockSpec((1, HID), lambda i: (0, 0)),
        ],
        out_specs=pl.BlockSpec((BE, HID), lambda i: (i, 0)),
        out_shape=jax.ShapeDtypeStruct((EP, HID), jnp.float32),
    )(edge_attr_p, w1, b1, wc, bc)


def _sc_mesh():
    return plsc.VectorSubcoreMesh(core_axis_name="c", subcore_axis_name="s")


_SC_PARAMS = pltpu.CompilerParams(needs_layout_passes=False,
                                  use_tc_tiling_on_sc=False)


def _pass1a(src2d, ef, hw, aux):
    NB = 3

    @functools.partial(
        pl.kernel,
        out_type=[jax.ShapeDtypeStruct((EP,), jnp.float32)],
        mesh=_sc_mesh(),
        compiler_params=_SC_PARAMS,
        scratch_types=(
            [pltpu.VMEM((K1 // 128, 128), jnp.int32) for _ in range(NB)] +
            [pltpu.VMEM((K1, HID), jnp.float32) for _ in range(NB)] +
            [pltpu.VMEM((K1,), jnp.float32) for _ in range(NB)] +
            [pltpu.VMEM((80,), jnp.float32)] +
            [pltpu.SemaphoreType.DMA for _ in range(4 * NB)]
        ),
    )
    def k(src_hbm, ef_hbm, hw_hbm, aux_hbm, logits_hbm, *bufs):
        srcbs = bufs[0:NB]
        efbs = bufs[NB:2 * NB]
        logbs = bufs[2 * NB:3 * NB]
        auxv = bufs[3 * NB]
        isems = bufs[3 * NB + 1:3 * NB + 1 + NB]
        esems = bufs[3 * NB + 1 + NB:3 * NB + 1 + 2 * NB]
        gsems = bufs[3 * NB + 1 + 2 * NB:3 * NB + 1 + 3 * NB]
        osems = bufs[3 * NB + 1 + 3 * NB:3 * NB + 1 + 4 * NB]
        c = lax.axis_index("c")
        s = lax.axis_index("s")
        wid = s * NC + c
        base = wid * EW1
        pltpu.sync_copy(aux_hbm, auxv)
        w2v = [auxv[pl.ds(q * L, L)] for q in range(HID // L)]
        b2s = auxv[pl.ds(HID, L)][0]

        def idx_desc(j, b):
            r = wid * (EW1 // 128) + j * (K1 // 128)
            return pltpu.make_async_copy(
                src_hbm.at[pl.ds(r, K1 // 128), :], srcbs[b], isems[b])

        def ef_desc(j, b):
            off = base + j * K1
            return pltpu.make_async_copy(
                ef_hbm.at[pl.ds(off, K1), :], efbs[b], esems[b])

        def g_start(b, g):
            pltpu.async_copy(hw_hbm.at[srcbs[b].at[g]],
                             efbs[b].at[pl.ds(g * 128, 128), :],
                             gsems[b], add=True)

        def g_wait(b, g):
            pltpu.make_async_copy(hw_hbm.at[srcbs[b].at[g]],
                                  efbs[b].at[pl.ds(g * 128, 128), :],
                                  gsems[b]).wait()

        def o_desc(j, b):
            off = base + j * K1
            return pltpu.make_async_copy(
                logbs[b], logits_hbm.at[pl.ds(off, K1)], osems[b])

        def fire_pre(j, b):
            idx_desc(j, b).start()
            ef_desc(j, b).start()

        def fire_gather(j, b):
            idx_desc(j, b).wait()
            ef_desc(j, b).wait()
            for g in range(K1 // 128):
                g_start(b, g)

        def do_compute(j, b):
            for g in range(K1 // 128):
                g_wait(b, g)

            def grp(i, carry2):
                lanes = jnp.arange(L, dtype=jnp.int32)
                lv = jnp.zeros((L,), jnp.float32)
                for u in range(L):
                    e = i * L + u
                    t = jnp.maximum(efbs[b][e, pl.ds(0, L)], 0.0) * w2v[0]
                    for q in range(1, HID // L):
                        t = t + jnp.maximum(efbs[b][e, pl.ds(q * L, L)],
                                            0.0) * w2v[q]
                    lv = lv + jnp.where(lanes == u, jnp.sum(t), 0.0)
                logbs[b][pl.ds(i * L, L)] = lv + b2s
                return carry2
            lax.fori_loop(0, K1 // L, grp, 0)
            o_desc(j, b).start()

        fire_pre(0, 0)
        fire_pre(1, 1)
        fire_gather(0, 0)

        def step3(jj, carry):
            for u in range(3):
                j = jj * 3 + u
                b = u

                @pl.when(j + 2 < C1)
                def _():
                    fire_pre(j + 2, (b + 2) % 3)

                @pl.when(j + 1 < C1)
                def _():
                    fire_gather(j + 1, (b + 1) % 3)

                @pl.when(j >= 3)
                def _():
                    o_desc(j - 3, b).wait()

                @pl.when(j < C1)
                def _():
                    do_compute(j, b)
            return carry
        lax.fori_loop(0, (C1 + 2) // 3, step3, 0)
        for j in range(((C1 + 2) // 3) * 3 - 3, C1):
            o_desc(j, j % 3).wait()

    return k(src2d, ef, hw, aux)


def _pass1b(dst2d, logits):
    @functools.partial(
        pl.kernel,
        out_type=[jax.ShapeDtypeStruct((NW, N2), jnp.float32)],
        mesh=_sc_mesh(),
        compiler_params=_SC_PARAMS,
        scratch_types=(
            [pltpu.VMEM((K1B // 128, 128), jnp.int32) for _ in range(2)] +
            [pltpu.VMEM((K1B,), jnp.float32) for _ in range(2)] +
            [pltpu.VMEM((N2,), jnp.float32)] +
            [pltpu.SemaphoreType.DMA for _ in range(4)]
        ),
    )
    def k(dst_hbm, logits_hbm, amax_hbm, db0, db1, lb0, lb1, amaxt,
          is0, is1, ls0, ls1):
        dbs, lbs = (db0, db1), (lb0, lb1)
        isems, lsems = (is0, is1), (ls0, ls1)
        c = lax.axis_index("c")
        s = lax.axis_index("s")
        wid = s * NC + c
        base = wid * EW1

        def initb(i, carry):
            amaxt[pl.ds(i * L, L)] = jnp.full((L,), NEG, jnp.float32)
            return carry
        lax.fori_loop(0, N2 // L, initb, 0)

        def idx_desc(j, b):
            r = wid * (EW1 // 128) + j * (K1B // 128)
            return pltpu.make_async_copy(
                dst_hbm.at[pl.ds(r, K1B // 128), :], dbs[b], isems[b])

        def log_desc(j, b):
            off = base + j * K1B
            return pltpu.make_async_copy(
                logits_hbm.at[pl.ds(off, K1B)], lbs[b], lsems[b])

        idx_desc(0, 0).start()
        log_desc(0, 0).start()

        def do_chunk(j, b):
            @pl.when(j + 1 < C1B)
            def _():
                idx_desc(j + 1, 1 - b).start()
                log_desc(j + 1, 1 - b).start()
            idx_desc(j, b).wait()
            log_desc(j, b).wait()

            def grp(i, carry2):
                row = i // 8
                col = (i % 8) * L
                dstv = dbs[b][row, pl.ds(col, L)]
                logit = lbs[b][pl.ds(i * L, L)]
                sd, sl = plsc.sort_key_val(dstv, logit)
                m = _seg_max_all(sd, sl)
                cur = plsc.load_gather(amaxt, [sd])
                plsc.store_scatter(amaxt, [sd], jnp.maximum(cur, m))
                return carry2
            lax.fori_loop(0, K1B // L, grp, 0)

        def step2(jj, carry):
            for u in range(2):
                j = jj * 2 + u

                @pl.when(j < C1B)
                def _():
                    do_chunk(j, u)
            return carry
        lax.fori_loop(0, (C1B + 1) // 2, step2, 0)
        pltpu.sync_copy(amaxt, amax_hbm.at[wid])

    return k(dst2d, logits)


def _amax_combine_body(parts_ref, out_ref):
    out_ref[...] = jnp.max(parts_ref[...], axis=0, keepdims=True)


def _amax_combine(parts):
    return pl.pallas_call(
        _amax_combine_body,
        grid=(N2 // 128,),
        in_specs=[pl.BlockSpec((NW, 128), lambda i: (0, i))],
        out_specs=pl.BlockSpec((1, 128), lambda i: (0, i)),
        out_shape=jax.ShapeDtypeStruct((1, N2), jnp.float32),
    )(parts)


def _pass2(src2d, dst2d, logits, amax, g2, zeros32, zeros1):
    NB = 3
    GG = K2 // 128

    @functools.partial(
        pl.kernel,
        out_type=[jax.ShapeDtypeStruct((NC, N2, HID // 2), jnp.float32),
                  jax.ShapeDtypeStruct((NC, N2), jnp.float32)],
        mesh=_sc_mesh(),
        compiler_params=_SC_PARAMS,
        scratch_types=(
            [pltpu.VMEM((GG, 128), jnp.int32) for _ in range(2 * NB)] +
            [pltpu.VMEM((K2,), jnp.float32) for _ in range(2 * NB)] +
            [pltpu.VMEM((K2, HID // 2), jnp.float32) for _ in range(NB)] +
            [pltpu.VMEM_SHARED((N2, HID // 2), jnp.float32),
             pltpu.VMEM_SHARED((N2,), jnp.float32)] +
            [pltpu.SemaphoreType.DMA for _ in range(3 * NB)]
        ),
    )
    def k(src_hbm, dst_hbm, logits_hbm, amax_hbm, g2_hbm, z32_hbm, z1_hbm,
          raw_hbm, asum_hbm, *bufs):
        srcbs = bufs[0:NB]
        dstbs = bufs[NB:2 * NB]
        logbs = bufs[2 * NB:3 * NB]
        amaxgs = bufs[3 * NB:4 * NB]
        growbs = bufs[4 * NB:5 * NB]
        accum = bufs[5 * NB]
        accum2 = bufs[5 * NB + 1]
        psems = bufs[5 * NB + 2:5 * NB + 2 + NB]
        gsems = bufs[5 * NB + 2 + NB:5 * NB + 2 + 2 * NB]
        ssems = bufs[5 * NB + 2 + 2 * NB:5 * NB + 2 + 3 * NB]
        c = lax.axis_index("c")
        s = lax.axis_index("s")
        pltpu.sync_copy(z32_hbm.at[pl.ds(s * NSL, NSL), :],
                        accum.at[pl.ds(s * NSL, NSL), :])
        pltpu.sync_copy(z1_hbm.at[pl.ds(s * NSL, NSL)],
                        accum2.at[pl.ds(s * NSL, NSL)])
        plsc.subcore_barrier()

        base = s * EW2

        def src_desc(j, b):
            r = s * (EW2 // 128) + j * GG
            return pltpu.make_async_copy(
                src_hbm.at[pl.ds(r, GG), :], srcbs[b], psems[b])

        def dst_desc(j, b):
            r = s * (EW2 // 128) + j * GG
            return pltpu.make_async_copy(
                dst_hbm.at[pl.ds(r, GG), :], dstbs[b], psems[b])

        def log_desc(j, b):
            off = base + j * K2
            return pltpu.make_async_copy(
                logits_hbm.at[pl.ds(off, K2)], logbs[b], psems[b])

        def grow_desc(b, g):
            return pltpu.make_async_copy(
                g2_hbm.at[srcbs[b].at[g]],
                growbs[b].at[pl.ds(g * 128, 128), :], gsems[b])

        def amax_desc(b, g):
            return pltpu.make_async_copy(
                amax_hbm.at[dstbs[b].at[g]],
                amaxgs[b].at[pl.ds(g * 128, 128)], gsems[b])

        def scat_row_desc(b, g):
            return pltpu.make_async_copy(
                growbs[b].at[pl.ds(g * 128, 128), :],
                accum.at[dstbs[b].at[g]], ssems[b])

        def scat_sum_desc(b, g):
            return pltpu.make_async_copy(
                logbs[b].at[pl.ds(g * 128, 128)],
                accum2.at[dstbs[b].at[g]], ssems[b])

        def scat_wait(b):
            for g in range(GG):
                scat_row_desc(b, g).wait()
                scat_sum_desc(b, g).wait()

        def fire_pre(j, b, first=False):
            if not first:
                @pl.when(j >= 3)
                def _():
                    scat_wait(b)
            src_desc(j, b).start()
            dst_desc(j, b).start()
            log_desc(j, b).start()

        def prep_gather(j, b):
            src_desc(j, b).wait()
            dst_desc(j, b).wait()
            log_desc(j, b).wait()

            def adj(i, carry2):
                row = i // 8
                col = (i % 8) * L
                srcbs[b][row, pl.ds(col, L)] = (
                    srcbs[b][row, pl.ds(col, L)] * 2 + c)
                return carry2
            lax.fori_loop(0, GG * (128 // L), adj, 0)
            for g in range(GG):
                grow_desc(b, g).start()
                amax_desc(b, g).start()

        def compute(j, b):
            for g in range(GG):
                grow_desc(b, g).wait()
                amax_desc(b, g).wait()

            def grp(i, carry2):
                logv = logbs[b][pl.ds(i * L, L)]
                am = amaxgs[b][pl.ds(i * L, L)]
                logbs[b][pl.ds(i * L, L)] = jnp.exp(logv - am)
                return carry2
            lax.fori_loop(0, K2 // L, grp, 0)

            def rower(i, carry2):
                axv = logbs[b][pl.ds(i * L, L)]
                for u in range(L):
                    e = i * L + u
                    sc = axv[u]
                    growbs[b][e, pl.ds(0, L)] = (
                        growbs[b][e, pl.ds(0, L)] * sc)
                    growbs[b][e, pl.ds(L, L)] = (
                        growbs[b][e, pl.ds(L, L)] * sc)
                return carry2
            lax.fori_loop(0, K2 // L, rower, 0)
            for g in range(GG):
                pltpu.async_copy(growbs[b].at[pl.ds(g * 128, 128), :],
                                 accum.at[dstbs[b].at[g]], ssems[b],
                                 add=True)
                pltpu.async_copy(logbs[b].at[pl.ds(g * 128, 128)],
                                 accum2.at[dstbs[b].at[g]], ssems[b],
                                 add=True)

        def step3(jj, carry):
            for u in range(3):
                j = jj * 3 + u
                b = u

                @pl.when(j < C2)
                def _():
                    fire_pre(j, b, first=True)
                    prep_gather(j, b)
                    compute(j, b)
                    scat_wait(b)
            return carry
        lax.fori_loop(0, (C2 + 2) // 3, step3, 0)
        plsc.subcore_barrier()

        pltpu.sync_copy(accum.at[pl.ds(s * NSL, NSL), :],
                        raw_hbm.at[c].at[pl.ds(s * NSL, NSL), :])
        pltpu.sync_copy(accum2.at[pl.ds(s * NSL, NSL)],
                        asum_hbm.at[c].at[pl.ds(s * NSL, NSL)])

    return k(src2d, dst2d, logits, amax, g2, zeros32, zeros1)


def _node_body(env_ref, h_ref, lice_ref, rawa_ref, rawb_ref, asum_ref,
               en_w1, en_b1, en_w2, en_b2, en_w3, en_b3,
               fec_w1, fec_b1, fec_w2, fec_b2, fec_w3, fec_b3,
               gru_wih, gru_bih, gru_whh, gru_bhh,
               de_w1, de_b1, de_w2, de_b2,
               at_w1b, sc_ref,
               h_out, lice_out, hw_out, g_out):
    env_t = env_ref[...]
    h = h_ref[...]
    lice = lice_ref[...]
    beta = sc_ref[0, 0]
    temp_sens = sc_ref[0, 1]

    tn = (env_t[:, 0:1] - 10.0) / 5.0
    f = jnp.maximum(tn * fec_w1[...].T + fec_b1[...], 0.0)
    f = jnp.maximum(jnp.dot(f, fec_w2[...].T,
                            preferred_element_type=jnp.float32) + fec_b2[...], 0.0)
    fec = _softplus(jnp.sum(f * fec_w3[...], axis=1, keepdims=True)
                    + fec_b3[...])

    asum = asum_ref[...]
    raw = jnp.concatenate([rawa_ref[...], rawb_ref[...]], axis=-1)
    pressure = (beta * raw) / (asum + 1e-8)

    x = jnp.maximum(jnp.dot(env_t, en_w1[...].T,
                            preferred_element_type=jnp.float32) + en_b1[...], 0.0)
    x = jnp.maximum(jnp.dot(x, en_w2[...].T,
                            preferred_element_type=jnp.float32) + en_b2[...], 0.0)
    env_enc = jnp.dot(x, en_w3[...].T,
                      preferred_element_type=jnp.float32) + en_b3[...]

    din = jnp.concatenate([env_enc + pressure, env_t, lice], axis=-1)
    gi = jnp.dot(din, gru_wih[...].T,
                 preferred_element_type=jnp.float32) + gru_bih[...]
    gh = jnp.dot(h, gru_whh[...].T,
                 preferred_element_type=jnp.float32) + gru_bhh[...]
    i_r, i_z, i_n = gi[:, :HID], gi[:, HID:2 * HID], gi[:, 2 * HID:]
    h_r, h_z, h_n = gh[:, :HID], gh[:, HID:2 * HID], gh[:, 2 * HID:]
    r = _sigmoid(i_r + h_r)
    z = _sigmoid(i_z + h_z)
    n = jnp.tanh(i_n + r * h_n)
    h_new = (1.0 - z) * n + z * h

    d = jnp.maximum(jnp.dot(h_new, de_w1[...].T,
                            preferred_element_type=jnp.float32) + de_b1[...], 0.0)
    lice_base = _softplus(jnp.dot(d, de_w2[...].T,
                                  preferred_element_type=jnp.float32) + de_b2[...])
    lice_new = lice_base * (1.0 + temp_sens * (fec - 1.0))

    h_out[...] = h_new
    lice_out[...] = lice_new
    hw_out[...] = jnp.dot(h_new, at_w1b[...].T,
                          preferred_element_type=jnp.float32)
    g_out[...] = lice_new[:, 0:1] * h_new


def _node_step(env_t, h, lice, rawa, rawb, asum, p, at_w1b, scalars):
    grid = N // BN
    full = lambda r, c: pl.BlockSpec((r, c), lambda i: (0, 0))
    row = lambda c: pl.BlockSpec((BN, c), lambda i: (i, 0))
    return pl.pallas_call(
        _node_body,
        grid=(grid,),
        in_specs=[
            row(ENV), row(HID), row(FEAT), row(HID // 2), row(HID // 2),
            row(1),
            full(HID, ENV), full(1, HID), full(HID, HID), full(1, HID),
            full(HID, HID), full(1, HID),
            full(32, 1), full(1, 32), full(32, 32), full(1, 32),
            full(1, 32), full(1, 1),
            full(3 * HID, HID + ENV + FEAT), full(1, 3 * HID),
            full(3 * HID, HID), full(1, 3 * HID),
            full(HID, HID), full(1, HID), full(FEAT, HID), full(1, FEAT),
            full(HID, HID), full(1, 2),
        ],
        out_specs=[row(HID), row(FEAT), row(HID), row(HID)],
        out_shape=[
            jax.ShapeDtypeStruct((N, HID), jnp.float32),
            jax.ShapeDtypeStruct((N, FEAT), jnp.float32),
            jax.ShapeDtypeStruct((N, HID), jnp.float32),
            jax.ShapeDtypeStruct((N, HID), jnp.float32),
        ],
    )(env_t, h, lice, rawa, rawb, asum,
      p['en_w1'], p['en_b1'][None], p['en_w2'], p['en_b2'][None],
      p['en_w3'], p['en_b3'][None],
      p['fec_w1'], p['fec_b1'][None], p['fec_w2'], p['fec_b2'][None],
      p['fec_w3'], p['fec_b3'][None],
      p['gru_wih'], p['gru_bih'][None], p['gru_whh'], p['gru_bhh'][None],
      p['de_w1'], p['de_b1'][None], p['de_w2'], p['de_b2'][None],
      at_w1b, scalars)


def kernel(env_sequence, edge_attr, initial_lice, params, edge_index, add_noise):
    p = params
    beta = jnp.exp(p['log_beta'])
    scalars = jnp.stack([beta, p['temp_sensitivity']])[None]

    at_w1a = p['at_w1'][:, :HID]
    at_w1b = p['at_w1'][:, HID:]
    wc = at_w1a @ p['ee_w2']
    bc = (p['ee_b2'] @ at_w1a.T + p['at_b1'])[None]
    aux = jnp.concatenate([p['at_w2'][0], p['at_b2'],
                           jnp.zeros((15,), jnp.float32)])

    src_p = jnp.pad(edge_index[0], (0, EP - E)).reshape(EP // 128, 128)
    dst_p = jnp.pad(edge_index[1], (0, EP - E),
                    constant_values=N).reshape(EP // 128, 128)
    ea_p = jnp.pad(edge_attr, ((0, EP - E), (0, 0)))
    ef_proj = _efproj(ea_p, p['ee_w1'], p['ee_b1'][None], wc, bc)

    h = jnp.zeros((N, HID), dtype=jnp.float32)
    lice = initial_lice
    hw = jnp.zeros((N, HID), dtype=jnp.float32)
    g = jnp.zeros((N, HID), dtype=jnp.float32)
    zeros32 = jnp.zeros((N2, HID // 2), dtype=jnp.float32)
    zeros1 = jnp.zeros((N2,), dtype=jnp.float32)

    traj = []
    for t in range(T):
        env_t = env_sequence[t]
        if t == 0:
            rawa = jnp.zeros((N2, HID // 2), dtype=jnp.float32)
            rawb = rawa
            asum = jnp.zeros((N2, 1), dtype=jnp.float32)
        else:
            logits = _pass1a(src_p, ef_proj, hw, aux)[0]
            amax_parts = _pass1b(dst_p, logits)[0]
            amax = _amax_combine(amax_parts).reshape(N2)
            g2 = g.reshape(2 * N, HID // 2)
            raw, asum_out = _pass2(src_p, dst_p, logits, amax, g2,
                                   zeros32, zeros1)
            asum = asum_out[0][:, None]
            rawa, rawb = raw[0], raw[1]
        h, lice, hw, g = _node_step(env_t, h, lice, rawa, rawb, asum, p,
                                    at_w1b, scalars)
        traj.append(lice)
    return jnp.stack(traj)


# speedup vs baseline: 14.9217x; 1.1074x over previous
import functools

import jax
import jax.numpy as jnp
from jax import lax
from jax.experimental import pallas as pl
from jax.experimental.pallas import tpu as pltpu
from jax.experimental.pallas import tpu_sc as plsc

N = 50000
E = 800000
T = 8
HID = 64
ENV = 5
FEAT = 3

NC, NS, L = 2, 16, 16
NW = NC * NS
EP = 819200
EW1 = EP // NW
K1 = 512
C1 = EW1 // K1
K1B = 1024
C1B = EW1 // K1B
EW2 = EP // NS
K2 = 256
C2 = EW2 // K2
N2 = 50048
NSL = N2 // NS
NEG = -3.0e38

BN = 1000
BE = 8192


def _softplus(x):
    return jnp.maximum(x, 0.0) + jnp.log1p(jnp.exp(-jnp.abs(x)))


def _sigmoid(x):
    return 1.0 / (1.0 + jnp.exp(-x))


def _take(x, i):
    dnums = lax.GatherDimensionNumbers(offset_dims=(),
                                       collapsed_slice_dims=(0,),
                                       start_index_map=(0,))
    return lax.gather(x, i[:, None], dnums, (1,),
                      mode=lax.GatherScatterMode.PROMISE_IN_BOUNDS)


def _seg_max_all(keys, vals):
    idx = jnp.arange(L, dtype=jnp.int32)
    v = vals
    for k in (1, 2, 4, 8):
        p = jnp.maximum(idx - k, 0)
        ok = (idx >= k) & (_take(keys, p) == keys)
        v = jnp.where(ok, jnp.maximum(v, _take(v, p)), v)
    for k in (1, 2, 4, 8):
        nx = jnp.minimum(idx + k, L - 1)
        ok = (idx <= L - 1 - k) & (_take(keys, nx) == keys)
        v = jnp.where(ok, jnp.maximum(v, _take(v, nx)), v)
    return v


def _seg_sum_all(keys, vals):
    idx = jnp.arange(L, dtype=jnp.int32)
    v = vals
    for k in (1, 2, 4, 8):
        p = jnp.maximum(idx - k, 0)
        ok = (idx >= k) & (_take(keys, p) == keys)
        v = v + jnp.where(ok, _take(v, p), 0.0)
    for k in (1, 2, 4, 8):
        nx = jnp.minimum(idx + k, L - 1)
        ok = (idx <= L - 1 - k) & (_take(keys, nx) == keys)
        v = jnp.where(ok, jnp.maximum(v, _take(v, nx)), v)
    return v


def _efproj_body(ea_ref, w1_ref, b1_ref, wc_ref, bc_ref, out_ref):
    ea = ea_ref[...]
    e1 = jnp.maximum(jnp.dot(ea, w1_ref[...].T,
                             preferred_element_type=jnp.float32) + b1_ref[...], 0.0)
    out_ref[...] = jnp.dot(e1, wc_ref[...].T,
                           preferred_element_type=jnp.float32) + bc_ref[...]


def _efproj(edge_attr_p, w1, b1, wc, bc):
    return pl.pallas_call(
        _efproj_body,
        grid=(EP // BE,),
        in_specs=[
            pl.BlockSpec((BE, 4), lambda i: (i, 0)),
            pl.BlockSpec((HID, 4), lambda i: (0, 0)),
            pl.BlockSpec((1, HID), lambda i: (0, 0)),
            pl.BlockSpec((HID, HID), lambda i: (0, 0)),
            pl.BlockSpec((1, HID), lambda i: (0, 0)),
        ],
        out_specs=pl.BlockSpec((BE, HID), lambda i: (i, 0)),
        out_shape=jax.ShapeDtypeStruct((EP, HID), jnp.float32),
    )(edge_attr_p, w1, b1, wc, bc)


def _sc_mesh():
    return plsc.VectorSubcoreMesh(core_axis_name="c", subcore_axis_name="s")


_SC_PARAMS = pltpu.CompilerParams(needs_layout_passes=False,
                                  use_tc_tiling_on_sc=False)


def _pass1a(src2d, ef, hw, aux):
    NB = 3

    @functools.partial(
        pl.kernel,
        out_type=[jax.ShapeDtypeStruct((EP,), jnp.float32)],
        mesh=_sc_mesh(),
        compiler_params=_SC_PARAMS,
        scratch_types=(
            [pltpu.VMEM((K1 // 128, 128), jnp.int32) for _ in range(NB)] +
            [pltpu.VMEM((K1, HID), jnp.float32) for _ in range(NB)] +
            [pltpu.VMEM((K1,), jnp.float32) for _ in range(NB)] +
            [pltpu.VMEM((80,), jnp.float32)] +
            [pltpu.SemaphoreType.DMA for _ in range(4 * NB)]
        ),
    )
    def k(src_hbm, ef_hbm, hw_hbm, aux_hbm, logits_hbm, *bufs):
        srcbs = bufs[0:NB]
        efbs = bufs[NB:2 * NB]
        logbs = bufs[2 * NB:3 * NB]
        auxv = bufs[3 * NB]
        isems = bufs[3 * NB + 1:3 * NB + 1 + NB]
        esems = bufs[3 * NB + 1 + NB:3 * NB + 1 + 2 * NB]
        gsems = bufs[3 * NB + 1 + 2 * NB:3 * NB + 1 + 3 * NB]
        osems = bufs[3 * NB + 1 + 3 * NB:3 * NB + 1 + 4 * NB]
        c = lax.axis_index("c")
        s = lax.axis_index("s")
        wid = s * NC + c
        base = wid * EW1
        pltpu.sync_copy(aux_hbm, auxv)
        w2v = [auxv[pl.ds(q * L, L)] for q in range(HID // L)]
        b2s = auxv[pl.ds(HID, L)][0]

        def idx_desc(j, b):
            r = wid * (EW1 // 128) + j * (K1 // 128)
            return pltpu.make_async_copy(
                src_hbm.at[pl.ds(r, K1 // 128), :], srcbs[b], isems[b])

        def ef_desc(j, b):
            off = base + j * K1
            return pltpu.make_async_copy(
                ef_hbm.at[pl.ds(off, K1), :], efbs[b], esems[b])

        def g_start(b, g):
            pltpu.async_copy(hw_hbm.at[srcbs[b].at[g]],
                             efbs[b].at[pl.ds(g * 128, 128), :],
                             gsems[b], add=True)

        def g_wait(b, g):
            pltpu.make_async_copy(hw_hbm.at[srcbs[b].at[g]],
                                  efbs[b].at[pl.ds(g * 128, 128), :],
                                  gsems[b]).wait()

        def o_desc(j, b):
            off = base + j * K1
            return pltpu.make_async_copy(
                logbs[b], logits_hbm.at[pl.ds(off, K1)], osems[b])

        def fire_pre(j, b):
            idx_desc(j, b).start()
            ef_desc(j, b).start()

        def fire_gather(j, b):
            idx_desc(j, b).wait()
            ef_desc(j, b).wait()
            for g in range(K1 // 128):
                g_start(b, g)

        def do_compute(j, b):
            for g in range(K1 // 128):
                g_wait(b, g)

            def grp(i, carry2):
                lanes = jnp.arange(L, dtype=jnp.int32)
                lv = jnp.zeros((L,), jnp.float32)
                for u in range(L):
                    e = i * L + u
                    t = jnp.maximum(efbs[b][e, pl.ds(0, L)], 0.0) * w2v[0]
                    for q in range(1, HID // L):
                        t = t + jnp.maximum(efbs[b][e, pl.ds(q * L, L)],
                                            0.0) * w2v[q]
                    lv = lv + jnp.where(lanes == u, jnp.sum(t), 0.0)
                logbs[b][pl.ds(i * L, L)] = lv + b2s
                return carry2
            lax.fori_loop(0, K1 // L, grp, 0)
            o_desc(j, b).start()

        fire_pre(0, 0)
        fire_pre(1, 1)
        fire_gather(0, 0)

        def step3(jj, carry):
            for u in range(3):
                j = jj * 3 + u
                b = u

                @pl.when(j + 2 < C1)
                def _():
                    fire_pre(j + 2, (b + 2) % 3)

                @pl.when(j + 1 < C1)
                def _():
                    fire_gather(j + 1, (b + 1) % 3)

                @pl.when(j >= 3)
                def _():
                    o_desc(j - 3, b).wait()

                @pl.when(j < C1)
                def _():
                    do_compute(j, b)
            return carry
        lax.fori_loop(0, (C1 + 2) // 3, step3, 0)
        for j in range(((C1 + 2) // 3) * 3 - 3, C1):
            o_desc(j, j % 3).wait()

    return k(src2d, ef, hw, aux)


def _pass1b(dst2d, logits):
    @functools.partial(
        pl.kernel,
        out_type=[jax.ShapeDtypeStruct((NW, N2), jnp.float32)],
        mesh=_sc_mesh(),
        compiler_params=_SC_PARAMS,
        scratch_types=(
            [pltpu.VMEM((K1B // 128, 128), jnp.int32) for _ in range(2)] +
            [pltpu.VMEM((K1B,), jnp.float32) for _ in range(2)] +
            [pltpu.VMEM((N2,), jnp.float32)] +
            [pltpu.SemaphoreType.DMA for _ in range(4)]
        ),
    )
    def k(dst_hbm, logits_hbm, amax_hbm, db0, db1, lb0, lb1, amaxt,
          is0, is1, ls0, ls1):
        dbs, lbs = (db0, db1), (lb0, lb1)
        isems, lsems = (is0, is1), (ls0, ls1)
        c = lax.axis_index("c")
        s = lax.axis_index("s")
        wid = s * NC + c
        base = wid * EW1

        def initb(i, carry):
            amaxt[pl.ds(i * L, L)] = jnp.full((L,), NEG, jnp.float32)
            return carry
        lax.fori_loop(0, N2 // L, initb, 0)

        def idx_desc(j, b):
            r = wid * (EW1 // 128) + j * (K1B // 128)
            return pltpu.make_async_copy(
                dst_hbm.at[pl.ds(r, K1B // 128), :], dbs[b], isems[b])

        def log_desc(j, b):
            off = base + j * K1B
            return pltpu.make_async_copy(
                logits_hbm.at[pl.ds(off, K1B)], lbs[b], lsems[b])

        idx_desc(0, 0).start()
        log_desc(0, 0).start()

        def do_chunk(j, b):
            @pl.when(j + 1 < C1B)
            def _():
                idx_desc(j + 1, 1 - b).start()
                log_desc(j + 1, 1 - b).start()
            idx_desc(j, b).wait()
            log_desc(j, b).wait()

            def grp(i, carry2):
                row = i // 8
                col = (i % 8) * L
                dstv = dbs[b][row, pl.ds(col, L)]
                logit = lbs[b][pl.ds(i * L, L)]
                sd, sl = plsc.sort_key_val(dstv, logit)
                m = _seg_max_all(sd, sl)
                cur = plsc.load_gather(amaxt, [sd])
                plsc.store_scatter(amaxt, [sd], jnp.maximum(cur, m))
                return carry2
            lax.fori_loop(0, K1B // L, grp, 0)

        def step2(jj, carry):
            for u in range(2):
                j = jj * 2 + u

                @pl.when(j < C1B)
                def _():
                    do_chunk(j, u)
            return carry
        lax.fori_loop(0, (C1B + 1) // 2, step2, 0)
        pltpu.sync_copy(amaxt, amax_hbm.at[wid])

    return k(dst2d, logits)


def _amax_combine_body(parts_ref, out_ref):
    out_ref[...] = jnp.max(parts_ref[...], axis=0, keepdims=True)


def _amax_combine(parts):
    return pl.pallas_call(
        _amax_combine_body,
        grid=(N2 // 128,),
        in_specs=[pl.BlockSpec((NW, 128), lambda i: (0, i))],
        out_specs=pl.BlockSpec((1, 128), lambda i: (0, i)),
        out_shape=jax.ShapeDtypeStruct((1, N2), jnp.float32),
    )(parts)


def _pass2(src2d, dst2d, logits, amax, g2, zeros32, zeros1):
    NB = 3
    GG = K2 // 128

    @functools.partial(
        pl.kernel,
        out_type=[jax.ShapeDtypeStruct((NC, N2, HID // 2), jnp.float32),
                  jax.ShapeDtypeStruct((NC, N2), jnp.float32)],
        mesh=_sc_mesh(),
        compiler_params=_SC_PARAMS,
        scratch_types=(
            [pltpu.VMEM((GG, 128), jnp.int32) for _ in range(2 * NB)] +
            [pltpu.VMEM((K2,), jnp.float32) for _ in range(2 * NB)] +
            [pltpu.VMEM((K2, HID // 2), jnp.float32) for _ in range(NB)] +
            [pltpu.VMEM_SHARED((N2, HID // 2), jnp.float32),
             pltpu.VMEM_SHARED((N2,), jnp.float32)] +
            [pltpu.SemaphoreType.DMA for _ in range(3 * NB)]
        ),
    )
    def k(src_hbm, dst_hbm, logits_hbm, amax_hbm, g2_hbm, z32_hbm, z1_hbm,
          raw_hbm, asum_hbm, *bufs):
        srcbs = bufs[0:NB]
        dstbs = bufs[NB:2 * NB]
        logbs = bufs[2 * NB:3 * NB]
        amaxgs = bufs[3 * NB:4 * NB]
        growbs = bufs[4 * NB:5 * NB]
        accum = bufs[5 * NB]
        accum2 = bufs[5 * NB + 1]
        psems = bufs[5 * NB + 2:5 * NB + 2 + NB]
        gsems = bufs[5 * NB + 2 + NB:5 * NB + 2 + 2 * NB]
        ssems = bufs[5 * NB + 2 + 2 * NB:5 * NB + 2 + 3 * NB]
        c = lax.axis_index("c")
        s = lax.axis_index("s")
        pltpu.sync_copy(z32_hbm.at[pl.ds(s * NSL, NSL), :],
                        accum.at[pl.ds(s * NSL, NSL), :])
        pltpu.sync_copy(z1_hbm.at[pl.ds(s * NSL, NSL)],
                        accum2.at[pl.ds(s * NSL, NSL)])
        plsc.subcore_barrier()

        base = s * EW2

        def src_desc(j, b):
            r = s * (EW2 // 128) + j * GG
            return pltpu.make_async_copy(
                src_hbm.at[pl.ds(r, GG), :], srcbs[b], psems[b])

        def dst_desc(j, b):
            r = s * (EW2 // 128) + j * GG
            return pltpu.make_async_copy(
                dst_hbm.at[pl.ds(r, GG), :], dstbs[b], psems[b])

        def log_desc(j, b):
            off = base + j * K2
            return pltpu.make_async_copy(
                logits_hbm.at[pl.ds(off, K2)], logbs[b], psems[b])

        def grow_desc(b, g):
            return pltpu.make_async_copy(
                g2_hbm.at[srcbs[b].at[g]],
                growbs[b].at[pl.ds(g * 128, 128), :], gsems[b])

        def amax_desc(b, g):
            return pltpu.make_async_copy(
                amax_hbm.at[dstbs[b].at[g]],
                amaxgs[b].at[pl.ds(g * 128, 128)], gsems[b])

        def scat_row_desc(b, g):
            return pltpu.make_async_copy(
                growbs[b].at[pl.ds(g * 128, 128), :],
                accum.at[dstbs[b].at[g]], ssems[b])

        def scat_sum_desc(b, g):
            return pltpu.make_async_copy(
                logbs[b].at[pl.ds(g * 128, 128)],
                accum2.at[dstbs[b].at[g]], ssems[b])

        def scat_wait(b):
            for g in range(GG):
                scat_row_desc(b, g).wait()
                scat_sum_desc(b, g).wait()

        def fire_pre(j, b, first=False):
            if not first:
                @pl.when(j >= 3)
                def _():
                    scat_wait(b)
            src_desc(j, b).start()
            dst_desc(j, b).start()
            log_desc(j, b).start()

        def prep_gather(j, b):
            src_desc(j, b).wait()
            dst_desc(j, b).wait()
            log_desc(j, b).wait()

            def adj(i, carry2):
                row = i // 8
                col = (i % 8) * L
                srcbs[b][row, pl.ds(col, L)] = (
                    srcbs[b][row, pl.ds(col, L)] * 2 + c)
                return carry2
            lax.fori_loop(0, GG * (128 // L), adj, 0)
            for g in range(GG):
                grow_desc(b, g).start()
                amax_desc(b, g).start()

        def compute(j, b):
            for g in range(GG):
                grow_desc(b, g).wait()
                amax_desc(b, g).wait()

            def grp(i, carry2):
                logv = logbs[b][pl.ds(i * L, L)]
                am = amaxgs[b][pl.ds(i * L, L)]
                logbs[b][pl.ds(i * L, L)] = jnp.exp(logv - am)
                return carry2
            lax.fori_loop(0, K2 // L, grp, 0)

            def rower(i, carry2):
                axv = logbs[b][pl.ds(i * L, L)]
                for u in range(L):
                    e = i * L + u
                    sc = axv[u]
                    growbs[b][e, pl.ds(0, L)] = (
                        growbs[b][e, pl.ds(0, L)] * sc)
                    growbs[b][e, pl.ds(L, L)] = (
                        growbs[b][e, pl.ds(L, L)] * sc)
                return carry2
            lax.fori_loop(0, K2 // L, rower, 0)
            for g in range(GG):
                pltpu.async_copy(growbs[b].at[pl.ds(g * 128, 128), :],
                                 accum.at[dstbs[b].at[g]], ssems[b],
                                 add=True)
                pltpu.async_copy(logbs[b].at[pl.ds(g * 128, 128)],
                                 accum2.at[dstbs[b].at[g]], ssems[b],
                                 add=True)

        fire_pre(0, 0, first=True)
        fire_pre(1, 1, first=True)
        prep_gather(0, 0)

        def step3(jj, carry):
            for u in range(3):
                j = jj * 3 + u
                b = u

                @pl.when(j + 2 < C2)
                def _():
                    fire_pre(j + 2, (b + 2) % 3)

                @pl.when(j + 1 < C2)
                def _():
                    prep_gather(j + 1, (b + 1) % 3)

                @pl.when(j < C2)
                def _():
                    compute(j, b)
            return carry
        lax.fori_loop(0, (C2 + 2) // 3, step3, 0)
        for j in (C2 - 3, C2 - 2, C2 - 1):
            scat_wait(j % 3)
        plsc.subcore_barrier()

        pltpu.sync_copy(accum.at[pl.ds(s * NSL, NSL), :],
                        raw_hbm.at[c].at[pl.ds(s * NSL, NSL), :])
        pltpu.sync_copy(accum2.at[pl.ds(s * NSL, NSL)],
                        asum_hbm.at[c].at[pl.ds(s * NSL, NSL)])

    return k(src2d, dst2d, logits, amax, g2, zeros32, zeros1)


def _node_body(env_ref, h_ref, lice_ref, rawa_ref, rawb_ref, asum_ref,
               en_w1, en_b1, en_w2, en_b2, en_w3, en_b3,
               fec_w1, fec_b1, fec_w2, fec_b2, fec_w3, fec_b3,
               gru_wih, gru_bih, gru_whh, gru_bhh,
               de_w1, de_b1, de_w2, de_b2,
               at_w1b, sc_ref,
               h_out, lice_out, hw_out, g_out):
    env_t = env_ref[...]
    h = h_ref[...]
    lice = lice_ref[...]
    beta = sc_ref[0, 0]
    temp_sens = sc_ref[0, 1]

    tn = (env_t[:, 0:1] - 10.0) / 5.0
    f = jnp.maximum(tn * fec_w1[...].T + fec_b1[...], 0.0)
    f = jnp.maximum(jnp.dot(f, fec_w2[...].T,
                            preferred_element_type=jnp.float32) + fec_b2[...], 0.0)
    fec = _softplus(jnp.sum(f * fec_w3[...], axis=1, keepdims=True)
                    + fec_b3[...])

    asum = asum_ref[...]
    raw = jnp.concatenate([rawa_ref[...], rawb_ref[...]], axis=-1)
    pressure = (beta * raw) / (asum + 1e-8)

    x = jnp.maximum(jnp.dot(env_t, en_w1[...].T,
                            preferred_element_type=jnp.float32) + en_b1[...], 0.0)
    x = jnp.maximum(jnp.dot(x, en_w2[...].T,
                            preferred_element_type=jnp.float32) + en_b2[...], 0.0)
    env_enc = jnp.dot(x, en_w3[...].T,
                      preferred_element_type=jnp.float32) + en_b3[...]

    din = jnp.concatenate([env_enc + pressure, env_t, lice], axis=-1)
    gi = jnp.dot(din, gru_wih[...].T,
                 preferred_element_type=jnp.float32) + gru_bih[...]
    gh = jnp.dot(h, gru_whh[...].T,
                 preferred_element_type=jnp.float32) + gru_bhh[...]
    i_r, i_z, i_n = gi[:, :HID], gi[:, HID:2 * HID], gi[:, 2 * HID:]
    h_r, h_z, h_n = gh[:, :HID], gh[:, HID:2 * HID], gh[:, 2 * HID:]
    r = _sigmoid(i_r + h_r)
    z = _sigmoid(i_z + h_z)
    n = jnp.tanh(i_n + r * h_n)
    h_new = (1.0 - z) * n + z * h

    d = jnp.maximum(jnp.dot(h_new, de_w1[...].T,
                            preferred_element_type=jnp.float32) + de_b1[...], 0.0)
    lice_base = _softplus(jnp.dot(d, de_w2[...].T,
                                  preferred_element_type=jnp.float32) + de_b2[...])
    lice_new = lice_base * (1.0 + temp_sens * (fec - 1.0))

    h_out[...] = h_new
    lice_out[...] = lice_new
    hw_out[...] = jnp.dot(h_new, at_w1b[...].T,
                          preferred_element_type=jnp.float32)
    g_out[...] = lice_new[:, 0:1] * h_new


def _node_step(env_t, h, lice, rawa, rawb, asum, p, at_w1b, scalars):
    grid = N // BN
    full = lambda r, c: pl.BlockSpec((r, c), lambda i: (0, 0))
    row = lambda c: pl.BlockSpec((BN, c), lambda i: (i, 0))
    return pl.pallas_call(
        _node_body,
        grid=(grid,),
        in_specs=[
            row(ENV), row(HID), row(FEAT), row(HID // 2), row(HID // 2),
            row(1),
            full(HID, ENV), full(1, HID), full(HID, HID), full(1, HID),
            full(HID, HID), full(1, HID),
            full(32, 1), full(1, 32), full(32, 32), full(1, 32),
            full(1, 32), full(1, 1),
            full(3 * HID, HID + ENV + FEAT), full(1, 3 * HID),
            full(3 * HID, HID), full(1, 3 * HID),
            full(HID, HID), full(1, HID), full(FEAT, HID), full(1, FEAT),
            full(HID, HID), full(1, 2),
        ],
        out_specs=[row(HID), row(FEAT), row(HID), row(HID)],
        out_shape=[
            jax.ShapeDtypeStruct((N, HID), jnp.float32),
            jax.ShapeDtypeStruct((N, FEAT), jnp.float32),
            jax.ShapeDtypeStruct((N, HID), jnp.float32),
            jax.ShapeDtypeStruct((N, HID), jnp.float32),
        ],
    )(env_t, h, lice, rawa, rawb, asum,
      p['en_w1'], p['en_b1'][None], p['en_w2'], p['en_b2'][None],
      p['en_w3'], p['en_b3'][None],
      p['fec_w1'], p['fec_b1'][None], p['fec_w2'], p['fec_b2'][None],
      p['fec_w3'], p['fec_b3'][None],
      p['gru_wih'], p['gru_bih'][None], p['gru_whh'], p['gru_bhh'][None],
      p['de_w1'], p['de_b1'][None], p['de_w2'], p['de_b2'][None],
      at_w1b, scalars)


def kernel(env_sequence, edge_attr, initial_lice, params, edge_index, add_noise):
    p = params
    beta = jnp.exp(p['log_beta'])
    scalars = jnp.stack([beta, p['temp_sensitivity']])[None]

    at_w1a = p['at_w1'][:, :HID]
    at_w1b = p['at_w1'][:, HID:]
    wc = at_w1a @ p['ee_w2']
    bc = (p['ee_b2'] @ at_w1a.T + p['at_b1'])[None]
    aux = jnp.concatenate([p['at_w2'][0], p['at_b2'],
                           jnp.zeros((15,), jnp.float32)])

    src_p = jnp.pad(edge_index[0], (0, EP - E)).reshape(EP // 128, 128)
    dst_p = jnp.pad(edge_index[1], (0, EP - E),
                    constant_values=N).reshape(EP // 128, 128)
    ea_p = jnp.pad(edge_attr, ((0, EP - E), (0, 0)))
    ef_proj = _efproj(ea_p, p['ee_w1'], p['ee_b1'][None], wc, bc)

    h = jnp.zeros((N, HID), dtype=jnp.float32)
    lice = initial_lice
    hw = jnp.zeros((N, HID), dtype=jnp.float32)
    g = jnp.zeros((N, HID), dtype=jnp.float32)
    zeros32 = jnp.zeros((N2, HID // 2), dtype=jnp.float32)
    zeros1 = jnp.zeros((N2,), dtype=jnp.float32)

    traj = []
    for t in range(T):
        env_t = env_sequence[t]
        if t == 0:
            rawa = jnp.zeros((N2, HID // 2), dtype=jnp.float32)
            rawb = rawa
            asum = jnp.zeros((N2, 1), dtype=jnp.float32)
        else:
            logits = _pass1a(src_p, ef_proj, hw, aux)[0]
            amax_parts = _pass1b(dst_p, logits)[0]
            amax = _amax_combine(amax_parts).reshape(N2)
            g2 = g.reshape(2 * N, HID // 2)
            raw, asum_out = _pass2(src_p, dst_p, logits, amax, g2,
                                   zeros32, zeros1)
            asum = asum_out[0][:, None]
            rawa, rawb = raw[0], raw[1]
        h, lice, hw, g = _node_step(env_t, h, lice, rawa, rawb, asum, p,
                                    at_w1b, scalars)
        traj.append(lice)
    return jnp.stack(traj)
